# baseline (speedup 1.0000x reference)
import jax
import jax.numpy as jnp
from jax.experimental import pallas as pl

N = 10000
E = 640000
NUM_GRAPHS = 512
NUM_LAYERS = 6


def _mlp_kernel(pooled_ref, fcw_ref, fcb_ref, fc2w_ref, fc2b_ref, out_ref):
    z = pooled_ref[...] @ fcw_ref[...] + fcb_ref[...]
    z = z * jnp.tanh(jax.nn.softplus(z))
    out_ref[...] = z @ fc2w_ref[...] + fc2b_ref[...]


def kernel(x, edge_index, edge_attr, batch, params):
    src = edge_index[0]
    dst = edge_index[1]
    loop = jnp.arange(N, dtype=src.dtype)
    src_a = jnp.concatenate([src, loop])
    dst_a = jnp.concatenate([dst, loop])
    ea_sum = jax.ops.segment_sum(edge_attr, dst, num_segments=N)
    cnt = jax.ops.segment_sum(jnp.ones((E,), jnp.float32), dst, num_segments=N)
    ea_loop = ea_sum / jnp.maximum(cnt, 1.0)[:, None]
    ea_a = jnp.concatenate([edge_attr, ea_loop], axis=0)
    a = params['prelu_a']
    h = x
    for l, p in enumerate(params['layers']):
        xl = h @ p['Wl']
        xr = h @ p['Wr']
        el = ea_a @ p['We']
        m = xl[src_a] + xr[dst_a] + el
        s = jax.nn.leaky_relu(m, 0.2) @ p['att']
        smax = jax.ops.segment_max(s, dst_a, num_segments=N)
        ex = jnp.exp(s - smax[dst_a])
        den = jax.ops.segment_sum(ex, dst_a, num_segments=N)
        alpha = ex / (den[dst_a] + 1e-16)
        h = jax.ops.segment_sum(alpha[:, None] * xl[src_a], dst_a, num_segments=N) + p['b']
        if l < NUM_LAYERS - 1:
            h = (h - p['bn_mean']) / jnp.sqrt(p['bn_var'] + 1e-5) * p['bn_gamma'] + p['bn_beta']
            h = jnp.where(h >= 0, h, a * h)
    pooled = jax.ops.segment_sum(h, batch, num_segments=NUM_GRAPHS)
    out = pl.pallas_call(
        _mlp_kernel,
        out_shape=jax.ShapeDtypeStruct((NUM_GRAPHS, 3), jnp.float32),
    )(pooled, params['fc_W'], params['fc_b'], params['fc2_W'], params['fc2_b'])
    return out


# speedup vs baseline: 4.3151x; 4.3151x over previous
import functools

import jax
import jax.numpy as jnp
from jax import lax
from jax.experimental import pallas as pl
from jax.experimental.pallas import tpu as pltpu
from jax.experimental.pallas import tpu_sc as plsc

N = 10000
E = 640000
D_IN = 48
D_EDGE = 16
HID = 64
OUT_GAT = 128
NUM_LAYERS = 6
NUM_GRAPHS = 512

N_PAD = 10240
E_PAD = E + N_PAD
CH = 1280
PAD_NODE = N

_MESH = plsc.VectorSubcoreMesh(core_axis_name="c", subcore_axis_name="s")
_SC_PARAMS = pltpu.CompilerParams(
    needs_layout_passes=False, use_tc_tiling_on_sc=False)

F32 = jnp.float32
I32 = jnp.int32


def _wid():
    return lax.axis_index("s") * 2 + lax.axis_index("c")


def _zero_1d(ref, n):
    def zb(g, c):
        ref[pl.ds(g * 16, 16)] = jnp.zeros((16,), F32)
        return c
    lax.fori_loop(0, n // 16, zb, 0)


@functools.partial(
    pl.kernel,
    out_type=jax.ShapeDtypeStruct((D_IN, N_PAD), F32),
    mesh=_MESH,
    compiler_params=_SC_PARAMS,
    scratch_types=[
        pltpu.VMEM((320, D_IN), F32),
        pltpu.VMEM((D_IN, 320), F32),
    ],
)
def _sc_transpose_x(x_hbm, xt_hbm, row_v, col_v):
    wid = _wid()
    r0 = wid * 320
    pltpu.sync_copy(x_hbm.at[pl.ds(r0, 320)], row_v)
    lanes = jax.lax.iota(I32, 16)

    def body(g, c):
        ridx = g * 16 + lanes
        for cc in range(D_IN):
            v = plsc.load_gather(row_v, [ridx, jnp.full((16,), cc, I32)])
            col_v[cc, pl.ds(g * 16, 16)] = v
        return c

    lax.fori_loop(0, 20, body, 0)
    pltpu.sync_copy(col_v, xt_hbm.at[:, pl.ds(r0, 320)])


@functools.partial(
    pl.kernel,
    out_type=jax.ShapeDtypeStruct((D_EDGE, E), F32),
    mesh=_MESH,
    compiler_params=_SC_PARAMS,
    scratch_types=[
        pltpu.VMEM((2000, D_EDGE), F32),
        pltpu.VMEM((D_EDGE, 2000), F32),
    ],
)
def _sc_transpose_ea(ea_hbm, eat_hbm, row_v, col_v):
    wid = _wid()
    e0 = wid * 20000
    lanes = jax.lax.iota(I32, 16)

    def chunk(k, c0):
        r0 = e0 + k * 2000
        pltpu.sync_copy(ea_hbm.at[pl.ds(r0, 2000)], row_v)

        def body(g, c):
            ridx = g * 16 + lanes
            for cc in range(D_EDGE):
                v = plsc.load_gather(row_v, [ridx, jnp.full((16,), cc, I32)])
                col_v[cc, pl.ds(g * 16, 16)] = v
            return c

        lax.fori_loop(0, 125, body, 0)
        pltpu.sync_copy(col_v, eat_hbm.at[:, pl.ds(r0, 2000)])
        return c0

    lax.fori_loop(0, 10, chunk, 0)


@functools.partial(
    pl.kernel,
    out_type=(
        jax.ShapeDtypeStruct((2, D_EDGE, N_PAD), F32),
        jax.ShapeDtypeStruct((2, N_PAD), F32),
    ),
    mesh=_MESH,
    compiler_params=_SC_PARAMS,
    scratch_types=[
        pltpu.VMEM((N_PAD,), F32),
        pltpu.VMEM((N_PAD,), F32),
        pltpu.VMEM((2000,), F32),
        pltpu.VMEM((2000,), I32),
    ],
)
def _sc_ea_segsum(eat_hbm, dst_hbm, sum_hbm, cnt_hbm, acc_v, cntacc_v, val_v,
                  dst_v):
    wid = _wid()
    col = wid % 16
    half = wid // 16
    e0 = half * (E // 2)
    _zero_1d(acc_v, N_PAD)
    _zero_1d(cntacc_v, N_PAD)
    ones = jnp.ones((16,), F32)

    def chunk(k, c0):
        r0 = e0 + k * 2000
        pltpu.sync_copy(eat_hbm.at[col, pl.ds(r0, 2000)], val_v)
        pltpu.sync_copy(dst_hbm.at[pl.ds(r0, 2000)], dst_v)

        def body(g, c):
            idx = dst_v[pl.ds(g * 16, 16)]
            v = val_v[pl.ds(g * 16, 16)]
            plsc.addupdate_scatter(acc_v, [idx], v)

            @pl.when(col == 0)
            def _():
                plsc.addupdate_scatter(cntacc_v, [idx], ones)

            return c

        lax.fori_loop(0, 125, body, 0)
        return c0

    lax.fori_loop(0, (E // 2) // 2000, chunk, 0)
    pltpu.sync_copy(acc_v, sum_hbm.at[half, col])

    @pl.when(col == 0)
    def _():
        pltpu.sync_copy(cntacc_v, cnt_hbm.at[half])


def _tc_ea_loop_body(sum_ref, cnt_ref, out_ref):
    s = sum_ref[0] + sum_ref[1]
    c = cnt_ref[0] + cnt_ref[1]
    out_ref[...] = s / jnp.maximum(c, 1.0)[None, :]


def _tc_ea_loop(sums, cnts):
    return pl.pallas_call(
        _tc_ea_loop_body,
        out_shape=jax.ShapeDtypeStruct((D_EDGE, N_PAD), F32),
    )(sums, cnts)


def _tc_proj_body(h_ref, wl_ref, wr_ref, xl_ref, xr_ref):
    h = h_ref[...]
    xl_ref[...] = jnp.dot(wl_ref[...], h, preferred_element_type=F32)
    xr_ref[...] = jnp.dot(wr_ref[...], h, preferred_element_type=F32)


def _tc_proj(hT, WlT, WrT):
    out = WlT.shape[0]
    return pl.pallas_call(
        _tc_proj_body,
        out_shape=(
            jax.ShapeDtypeStruct((out, N_PAD), F32),
            jax.ShapeDtypeStruct((out, N_PAD), F32),
        ),
    )(hT, WlT, WrT)


def _tc_act_body(parts_ref, sc_ref, sh_ref, a_ref, out_ref, *, es, bn):
    h = parts_ref[0]
    for i in range(1, es):
        h = h + parts_ref[i]
    if bn:
        h = h * sc_ref[...][:, None] + sh_ref[...][:, None]
        a = a_ref[0, 0]
        h = jnp.where(h >= 0, h, a * h)
    else:
        h = h + sc_ref[...][:, None]
    mask = lax.broadcasted_iota(I32, h.shape, 1) < N
    out_ref[...] = jnp.where(mask, h, 0.0)


def _tc_act(h_parts, scale, shift, a2d, es, bn):
    out = h_parts.shape[1]
    body = functools.partial(_tc_act_body, es=es, bn=bn)
    return pl.pallas_call(
        body,
        out_shape=jax.ShapeDtypeStruct((out, N_PAD), F32),
    )(h_parts, scale, shift, a2d)


def _tc_el_body(we_ref, ea_ref, out_ref):
    out_ref[...] = jnp.dot(we_ref[...], ea_ref[...], preferred_element_type=F32)


def _tc_el(WeT, ea_aT):
    out = WeT.shape[0]
    blk = 5120
    grid = E_PAD // blk
    return pl.pallas_call(
        _tc_el_body,
        grid=(grid,),
        in_specs=[
            pl.BlockSpec((out, D_EDGE), lambda i: (0, 0)),
            pl.BlockSpec((D_EDGE, blk), lambda i: (0, i)),
        ],
        out_specs=pl.BlockSpec((out, blk), lambda i: (0, i)),
        out_shape=jax.ShapeDtypeStruct((out, E_PAD), F32),
    )(WeT, ea_aT)


def _make_sc_phase_a(out_dim, cg, es):
    el_per = E_PAD // es
    nchunk = el_per // CH

    @functools.partial(
        pl.kernel,
        out_type=jax.ShapeDtypeStruct((cg, E_PAD), F32),
        mesh=_MESH,
        compiler_params=_SC_PARAMS,
        scratch_types=[
            pltpu.VMEM((4, N_PAD), F32),
            pltpu.VMEM((4, N_PAD), F32),
            pltpu.VMEM((4, 16), F32),
            pltpu.VMEM((CH,), I32),
            pltpu.VMEM((CH,), I32),
            pltpu.VMEM((4, CH), F32),
            pltpu.VMEM((CH,), F32),
        ],
    )
    def phase_a(xlt_hbm, xrt_hbm, elt_hbm, attbc_hbm, src_hbm, dst_hbm,
                sp_hbm, xl_v, xr_v, att_v, src_v, dst_v, el_v, s_v):
        wid = _wid()
        cgi = wid % cg
        esi = wid // cg
        e0 = esi * el_per
        for c in range(4):
            col = cgi * 4 + c
            pltpu.sync_copy(xlt_hbm.at[col], xl_v.at[c])
            pltpu.sync_copy(xrt_hbm.at[col], xr_v.at[c])
            pltpu.sync_copy(attbc_hbm.at[col], att_v.at[c])
        atts = tuple(att_v[c] for c in range(4))

        def chunk(k, c0):
            r0 = e0 + k * CH
            pltpu.sync_copy(src_hbm.at[pl.ds(r0, CH)], src_v)
            pltpu.sync_copy(dst_hbm.at[pl.ds(r0, CH)], dst_v)
            for c in range(4):
                col = cgi * 4 + c
                pltpu.sync_copy(elt_hbm.at[col, pl.ds(r0, CH)], el_v.at[c])

            def body(g, c1):
                idxs = src_v[pl.ds(g * 16, 16)]
                idxd = dst_v[pl.ds(g * 16, 16)]
                acc = jnp.zeros((16,), F32)
                for c in range(4):
                    xa = plsc.load_gather(xl_v.at[c], [idxs])
                    xb = plsc.load_gather(xr_v.at[c], [idxd])
                    m = xa + xb + el_v[c, pl.ds(g * 16, 16)]
                    m = jnp.maximum(m, 0.2 * m)
                    acc = acc + atts[c] * m
                s_v[pl.ds(g * 16, 16)] = acc
                return c1

            lax.fori_loop(0, CH // 16, body, 0)
            pltpu.sync_copy(s_v, sp_hbm.at[cgi, pl.ds(r0, CH)])
            return c0

        lax.fori_loop(0, nchunk, chunk, 0)

    return phase_a


def _tc_reduce_body(sp_ref, s_ref, g_ref):
    blk = sp_ref[...]
    ssum = blk.sum(axis=0)
    s_ref[...] = ssum
    bm = jnp.max(ssum)

    @pl.when(pl.program_id(0) == 0)
    def _():
        g_ref[...] = jnp.full((1, 128), -1e30, F32)

    g_ref[...] = jnp.maximum(g_ref[...], bm)


def _tc_reduce_s(s_parts3):
    cg = s_parts3.shape[0]
    rows = E_PAD // 128
    rblk = 8
    grid = rows // rblk
    return pl.pallas_call(
        _tc_reduce_body,
        grid=(grid,),
        in_specs=[pl.BlockSpec((cg, rblk, 128), lambda i: (0, i, 0))],
        out_specs=(
            pl.BlockSpec((rblk, 128), lambda i: (i, 0)),
            pl.BlockSpec((1, 128), lambda i: (0, 0)),
        ),
        out_shape=(
            jax.ShapeDtypeStruct((rows, 128), F32),
            jax.ShapeDtypeStruct((1, 128), F32),
        ),
    )(s_parts3)


_B2_PER = E_PAD // 32
_B2_CH = 2032


@functools.partial(
    pl.kernel,
    out_type=(
        jax.ShapeDtypeStruct((E_PAD,), F32),
        jax.ShapeDtypeStruct((32, N_PAD), F32),
    ),
    mesh=_MESH,
    compiler_params=_SC_PARAMS,
    scratch_types=[
        pltpu.VMEM((N_PAD,), F32),
        pltpu.VMEM((_B2_CH,), F32),
        pltpu.VMEM((_B2_CH,), I32),
        pltpu.VMEM((16,), F32),
    ],
)
def _sc_phase_b2(s_hbm, g_hbm, dst_hbm, ex_hbm, dp_hbm, den_v, s_v, dst_v,
                 g_v):
    wid = _wid()
    e0 = wid * _B2_PER
    pltpu.sync_copy(g_hbm.at[0, pl.ds(0, 16)], g_v)
    gv = g_v[...]
    _zero_1d(den_v, N_PAD)

    def chunk(k, c0):
        r0 = e0 + k * _B2_CH
        pltpu.sync_copy(s_hbm.at[pl.ds(r0, _B2_CH)], s_v)
        pltpu.sync_copy(dst_hbm.at[pl.ds(r0, _B2_CH)], dst_v)

        def body(g, c1):
            sv = s_v[pl.ds(g * 16, 16)]
            ex = jnp.exp(sv - gv)
            s_v[pl.ds(g * 16, 16)] = ex
            idx = dst_v[pl.ds(g * 16, 16)]
            plsc.addupdate_scatter(den_v, [idx], ex)
            return c1

        lax.fori_loop(0, _B2_CH // 16, body, 0)
        pltpu.sync_copy(s_v, ex_hbm.at[pl.ds(r0, _B2_CH)])
        return c0

    lax.fori_loop(0, _B2_PER // _B2_CH, chunk, 0)
    pltpu.sync_copy(den_v, dp_hbm.at[wid])


def _tc_rden_body(dp_ref, out_ref):
    out_ref[...] = 1.0 / (dp_ref[...].sum(axis=0) + 1e-16)


def _tc_rden(dp3):
    return pl.pallas_call(
        _tc_rden_body,
        out_shape=jax.ShapeDtypeStruct((N_PAD // 128, 128), F32),
    )(dp3)


def _make_sc_phase_c(out_dim, cg, es):
    el_per = E_PAD // es
    nchunk = el_per // CH

    @functools.partial(
        pl.kernel,
        out_type=jax.ShapeDtypeStruct((es, out_dim, N_PAD), F32),
        mesh=_MESH,
        compiler_params=_SC_PARAMS,
        scratch_types=[
            pltpu.VMEM((4, N_PAD), F32),
            pltpu.VMEM((4, N_PAD), F32),
            pltpu.VMEM((N_PAD,), F32),
            pltpu.VMEM((CH,), I32),
            pltpu.VMEM((CH,), I32),
            pltpu.VMEM((CH,), F32),
        ],
    )
    def phase_c(xlt_hbm, rden_hbm, ex_hbm, src_hbm, dst_hbm, hp_hbm,
                xl_v, h_v, rden_v, src_v, dst_v, ex_v):
        wid = _wid()
        cgi = wid % cg
        esi = wid // cg
        e0 = esi * el_per
        pltpu.sync_copy(rden_hbm, rden_v)
        for c in range(4):
            col = cgi * 4 + c
            pltpu.sync_copy(xlt_hbm.at[col], xl_v.at[c])

        def zb(g, c):
            z = jnp.zeros((16,), F32)
            for cc in range(4):
                h_v[cc, pl.ds(g * 16, 16)] = z
            return c

        lax.fori_loop(0, N_PAD // 16, zb, 0)

        def chunk(k, c0):
            r0 = e0 + k * CH
            pltpu.sync_copy(src_hbm.at[pl.ds(r0, CH)], src_v)
            pltpu.sync_copy(dst_hbm.at[pl.ds(r0, CH)], dst_v)
            pltpu.sync_copy(ex_hbm.at[pl.ds(r0, CH)], ex_v)

            def body(g, c1):
                idxs = src_v[pl.ds(g * 16, 16)]
                idxd = dst_v[pl.ds(g * 16, 16)]
                alpha = ex_v[pl.ds(g * 16, 16)] * plsc.load_gather(
                    rden_v, [idxd])
                for c in range(4):
                    xv = plsc.load_gather(xl_v.at[c], [idxs])
                    plsc.addupdate_scatter(h_v.at[c], [idxd], alpha * xv)
                return c1

            lax.fori_loop(0, CH // 16, body, 0)
            return c0

        lax.fori_loop(0, nchunk, chunk, 0)
        for c in range(4):
            col = cgi * 4 + c
            pltpu.sync_copy(h_v.at[c], hp_hbm.at[esi, col])

    return phase_c


@functools.partial(
    pl.kernel,
    out_type=jax.ShapeDtypeStruct((OUT_GAT, NUM_GRAPHS), F32),
    mesh=_MESH,
    compiler_params=_SC_PARAMS,
    scratch_types=[
        pltpu.VMEM((N_PAD,), I32),
        pltpu.VMEM((N_PAD,), F32),
        pltpu.VMEM((1024,), F32),
        pltpu.VMEM((16,), F32),
    ],
)
def _pool_sc(hT_hbm, batch_hbm, bbc_hbm, out_hbm, batch_v, col_v, pool_v, b_v):
    wid = _wid()
    pltpu.sync_copy(batch_hbm, batch_v)
    for c in range(4):
        col = wid * 4 + c
        pltpu.sync_copy(hT_hbm.at[col], col_v)
        pltpu.sync_copy(bbc_hbm.at[col], b_v)
        bv = b_v[...]
        _zero_1d(pool_v, 1024)

        def body(i, carry):
            v = col_v[pl.ds(i * 16, 16)] + bv
            idx = batch_v[pl.ds(i * 16, 16)]
            plsc.addupdate_scatter(pool_v, [idx], v)
            return carry

        lax.fori_loop(0, N_PAD // 16, body, 0)
        pltpu.sync_copy(pool_v.at[pl.ds(0, NUM_GRAPHS)], out_hbm.at[col])


def _tc_mlp_body(p_ref, w1_ref, b1_ref, w2_ref, b2_ref, out_ref):
    z = jnp.dot(w1_ref[...], p_ref[...], preferred_element_type=F32)
    z = z + b1_ref[...][:, None]
    z = z * jnp.tanh(jax.nn.softplus(z))
    o = jnp.dot(w2_ref[...], z, preferred_element_type=F32)
    out_ref[...] = o + b2_ref[...][:, None]


def _tc_mlp(pooledT, fcWT, fcb, fc2WT, fc2b):
    return pl.pallas_call(
        _tc_mlp_body,
        out_shape=jax.ShapeDtypeStruct((3, NUM_GRAPHS), F32),
    )(pooledT, fcWT, fcb, fc2WT, fc2b)


_PHASE_A_64 = _make_sc_phase_a(HID, 16, 2)
_PHASE_A_128 = _make_sc_phase_a(OUT_GAT, 32, 1)
_PHASE_C_64 = _make_sc_phase_c(HID, 16, 2)
_PHASE_C_128 = _make_sc_phase_c(OUT_GAT, 32, 1)


def kernel(x, edge_index, edge_attr, batch, params):
    src = edge_index[0]
    dst = edge_index[1]
    loop = jnp.arange(N, dtype=I32)
    padi = jnp.full((N_PAD - N,), PAD_NODE, I32)
    src_pad = jnp.concatenate([src, loop, padi])
    dst_pad = jnp.concatenate([dst, loop, padi])
    batch_pad = jnp.full((N_PAD,), NUM_GRAPHS, I32).at[:N].set(batch)
    x_pad = jnp.zeros((N_PAD, D_IN), F32).at[:N].set(x)

    xT = _sc_transpose_x(x_pad)
    eaT = _sc_transpose_ea(edge_attr)
    sums, cnts = _sc_ea_segsum(eaT, dst)
    ea_loopT = _tc_ea_loop(sums, cnts)
    ea_aT = jnp.concatenate([eaT, ea_loopT], axis=1)

    a2d = jnp.reshape(params['prelu_a'], (1, 1)).astype(F32)
    hT = xT
    for l, p in enumerate(params['layers']):
        out_dim = p['Wl'].shape[1]
        WlT = p['Wl'].T
        WrT = p['Wr'].T
        WeT = p['We'].T
        attbc = jnp.broadcast_to(p['att'][:, None], (out_dim, 16))
        xlT, xrT = _tc_proj(hT, WlT, WrT)
        elT = _tc_el(WeT, ea_aT)
        if out_dim == HID:
            cg, es = 16, 2
            sp = _PHASE_A_64(xlT, xrT, elT, attbc, src_pad, dst_pad)
        else:
            cg, es = 32, 1
            sp = _PHASE_A_128(xlT, xrT, elT, attbc, src_pad, dst_pad)
        s2d, gmax = _tc_reduce_s(sp.reshape(cg, E_PAD // 128, 128))
        s1 = s2d.reshape(E_PAD)
        ex, dparts = _sc_phase_b2(s1, gmax, dst_pad)
        rden2 = _tc_rden(dparts.reshape(32, N_PAD // 128, 128))
        rden = rden2.reshape(N_PAD)
        if out_dim == HID:
            h_parts = _PHASE_C_64(xlT, rden, ex, src_pad, dst_pad)
        else:
            h_parts = _PHASE_C_128(xlT, rden, ex, src_pad, dst_pad)
        if l < NUM_LAYERS - 1:
            scale = p['bn_gamma'] / jnp.sqrt(p['bn_var'] + 1e-5)
            shift = p['bn_beta'] - p['bn_mean'] * scale + p['b'] * scale
            hT = _tc_act(h_parts, scale, shift, a2d, es, True)
        else:
            hT128 = h_parts[0]
            bbc = jnp.broadcast_to(p['b'][:, None], (OUT_GAT, 16))
            pooledT = _pool_sc(hT128, batch_pad, bbc)

    out = _tc_mlp(pooledT, params['fc_W'].T, params['fc_b'],
                  params['fc2_W'].T, params['fc2_b'])
    return out.T


# speedup vs baseline: 6.3086x; 1.4620x over previous
import functools

import jax
import jax.numpy as jnp
from jax import lax
from jax.experimental import pallas as pl
from jax.experimental.pallas import tpu as pltpu
from jax.experimental.pallas import tpu_sc as plsc

N = 10000
E = 640000
D_IN = 48
D_EDGE = 16
HID = 64
OUT_GAT = 128
NUM_LAYERS = 6
NUM_GRAPHS = 512

N_PAD = 10240
E_PAD = E + N_PAD
CH = 4064
PAD_NODE = N

_MESH = plsc.VectorSubcoreMesh(core_axis_name="c", subcore_axis_name="s")
_SC_PARAMS = pltpu.CompilerParams(
    needs_layout_passes=False, use_tc_tiling_on_sc=False)

F32 = jnp.float32
I32 = jnp.int32


def _wid():
    return lax.axis_index("s") * 2 + lax.axis_index("c")


def _zero_1d(ref, n):
    def zb(g, c):
        ref[pl.ds(g * 16, 16)] = jnp.zeros((16,), F32)
        return c
    lax.fori_loop(0, n // 16, zb, 0)


@functools.partial(
    pl.kernel,
    out_type=jax.ShapeDtypeStruct((D_IN, N_PAD), F32),
    mesh=_MESH,
    compiler_params=_SC_PARAMS,
    scratch_types=[
        pltpu.VMEM((320, D_IN), F32),
        pltpu.VMEM((D_IN, 320), F32),
    ],
)
def _sc_transpose_x(x_hbm, xt_hbm, row_v, col_v):
    wid = _wid()
    r0 = wid * 320
    pltpu.sync_copy(x_hbm.at[pl.ds(r0, 320)], row_v)
    lanes = jax.lax.iota(I32, 16)

    def body(g, c):
        ridx = g * 16 + lanes
        for cc in range(D_IN):
            v = plsc.load_gather(row_v, [ridx, jnp.full((16,), cc, I32)])
            col_v[cc, pl.ds(g * 16, 16)] = v
        return c

    lax.fori_loop(0, 20, body, 0)
    pltpu.sync_copy(col_v, xt_hbm.at[:, pl.ds(r0, 320)])


@functools.partial(
    pl.kernel,
    out_type=jax.ShapeDtypeStruct((D_EDGE, E), F32),
    mesh=_MESH,
    compiler_params=_SC_PARAMS,
    scratch_types=[
        pltpu.VMEM((2000, D_EDGE), F32),
        pltpu.VMEM((D_EDGE, 2000), F32),
    ],
)
def _sc_transpose_ea(ea_hbm, eat_hbm, row_v, col_v):
    wid = _wid()
    e0 = wid * 20000
    lanes = jax.lax.iota(I32, 16)

    def chunk(k, c0):
        r0 = e0 + k * 2000
        pltpu.sync_copy(ea_hbm.at[pl.ds(r0, 2000)], row_v)

        def body(g, c):
            ridx = g * 16 + lanes
            for cc in range(D_EDGE):
                v = plsc.load_gather(row_v, [ridx, jnp.full((16,), cc, I32)])
                col_v[cc, pl.ds(g * 16, 16)] = v
            return c

        lax.fori_loop(0, 125, body, 0)
        pltpu.sync_copy(col_v, eat_hbm.at[:, pl.ds(r0, 2000)])
        return c0

    lax.fori_loop(0, 10, chunk, 0)


@functools.partial(
    pl.kernel,
    out_type=(
        jax.ShapeDtypeStruct((2, D_EDGE, N_PAD), F32),
        jax.ShapeDtypeStruct((2, N_PAD), F32),
    ),
    mesh=_MESH,
    compiler_params=_SC_PARAMS,
    scratch_types=[
        pltpu.VMEM((N_PAD,), F32),
        pltpu.VMEM((N_PAD,), F32),
        pltpu.VMEM((2000,), F32),
        pltpu.VMEM((2000,), I32),
    ],
)
def _sc_ea_segsum(eat_hbm, dst_hbm, sum_hbm, cnt_hbm, acc_v, cntacc_v, val_v,
                  dst_v):
    wid = _wid()
    col = wid % 16
    half = wid // 16
    e0 = half * (E // 2)
    _zero_1d(acc_v, N_PAD)
    _zero_1d(cntacc_v, N_PAD)
    ones = jnp.ones((16,), F32)

    def chunk(k, c0):
        r0 = e0 + k * 2000
        pltpu.sync_copy(eat_hbm.at[col, pl.ds(r0, 2000)], val_v)
        pltpu.sync_copy(dst_hbm.at[pl.ds(r0, 2000)], dst_v)

        def body(g, c):
            idx = dst_v[pl.ds(g * 16, 16)]
            v = val_v[pl.ds(g * 16, 16)]
            plsc.addupdate_scatter(acc_v, [idx], v)

            @pl.when(col == 0)
            def _():
                plsc.addupdate_scatter(cntacc_v, [idx], ones)

            return c

        lax.fori_loop(0, 125, body, 0)
        return c0

    lax.fori_loop(0, (E // 2) // 2000, chunk, 0)
    pltpu.sync_copy(acc_v, sum_hbm.at[half, col])

    @pl.when(col == 0)
    def _():
        pltpu.sync_copy(cntacc_v, cnt_hbm.at[half])


def _tc_ea_loop_body(sum_ref, cnt_ref, out_ref):
    s = sum_ref[0] + sum_ref[1]
    c = cnt_ref[0] + cnt_ref[1]
    out_ref[...] = s / jnp.maximum(c, 1.0)[None, :]


def _tc_ea_loop(sums, cnts):
    return pl.pallas_call(
        _tc_ea_loop_body,
        out_shape=jax.ShapeDtypeStruct((D_EDGE, N_PAD), F32),
    )(sums, cnts)


def _tc_proj_body(h_ref, wl_ref, wr_ref, xl_ref, xr_ref):
    h = h_ref[...]
    xl_ref[...] = jnp.dot(wl_ref[...], h, preferred_element_type=F32)
    xr_ref[...] = jnp.dot(wr_ref[...], h, preferred_element_type=F32)


def _tc_proj(hT, WlT, WrT):
    out = WlT.shape[0]
    return pl.pallas_call(
        _tc_proj_body,
        out_shape=(
            jax.ShapeDtypeStruct((out, N_PAD), F32),
            jax.ShapeDtypeStruct((out, N_PAD), F32),
        ),
    )(hT, WlT, WrT)


def _tc_act_body(parts_ref, sc_ref, sh_ref, a_ref, out_ref, *, es, bn):
    h = parts_ref[0]
    for i in range(1, es):
        h = h + parts_ref[i]
    if bn:
        h = h * sc_ref[...][:, None] + sh_ref[...][:, None]
        a = a_ref[0, 0]
        h = jnp.where(h >= 0, h, a * h)
    else:
        h = h + sc_ref[...][:, None]
    mask = lax.broadcasted_iota(I32, h.shape, 1) < N
    out_ref[...] = jnp.where(mask, h, 0.0)


def _tc_act(h_parts, scale, shift, a2d, es, bn):
    out = h_parts.shape[1]
    body = functools.partial(_tc_act_body, es=es, bn=bn)
    return pl.pallas_call(
        body,
        out_shape=jax.ShapeDtypeStruct((out, N_PAD), F32),
    )(h_parts, scale, shift, a2d)


def _tc_el_body(we_ref, ea_ref, out_ref):
    out_ref[...] = jnp.dot(we_ref[...], ea_ref[...], preferred_element_type=F32)


def _tc_el(WeT, ea_aT):
    out = WeT.shape[0]
    blk = 5120
    grid = E_PAD // blk
    return pl.pallas_call(
        _tc_el_body,
        grid=(grid,),
        in_specs=[
            pl.BlockSpec((out, D_EDGE), lambda i: (0, 0)),
            pl.BlockSpec((D_EDGE, blk), lambda i: (0, i)),
        ],
        out_specs=pl.BlockSpec((out, blk), lambda i: (0, i)),
        out_shape=jax.ShapeDtypeStruct((out, E_PAD), F32),
    )(WeT, ea_aT)


def _make_sc_phase_a(out_dim, cg, es):
    el_per = E_PAD // es
    nchunk = el_per // CH

    @functools.partial(
        pl.kernel,
        out_type=jax.ShapeDtypeStruct((cg, E_PAD), F32),
        mesh=_MESH,
        compiler_params=_SC_PARAMS,
        scratch_types=[
            pltpu.VMEM((4, N_PAD), F32),
            pltpu.VMEM((4, N_PAD), F32),
            pltpu.VMEM((4, 16), F32),
            pltpu.VMEM((CH,), I32),
            pltpu.VMEM((CH,), I32),
            pltpu.VMEM((4, CH), F32),
            pltpu.VMEM((CH,), F32),
        ],
    )
    def phase_a(xlt_hbm, xrt_hbm, elt_hbm, attbc_hbm, src_hbm, dst_hbm,
                sp_hbm, xl_v, xr_v, att_v, src_v, dst_v, el_v, s_v):
        wid = _wid()
        cgi = wid % cg
        esi = wid // cg
        e0 = esi * el_per
        c4 = cgi * 4
        pltpu.sync_copy(xlt_hbm.at[pl.ds(c4, 4)], xl_v)
        pltpu.sync_copy(xrt_hbm.at[pl.ds(c4, 4)], xr_v)
        pltpu.sync_copy(attbc_hbm.at[pl.ds(c4, 4)], att_v)
        atts = tuple(att_v[c] for c in range(4))

        def chunk(k, c0):
            r0 = e0 + k * CH
            pltpu.sync_copy(src_hbm.at[pl.ds(r0, CH)], src_v)
            pltpu.sync_copy(dst_hbm.at[pl.ds(r0, CH)], dst_v)
            pltpu.sync_copy(elt_hbm.at[pl.ds(c4, 4), pl.ds(r0, CH)], el_v)

            def body(g, c1):
                idxs = src_v[pl.ds(g * 16, 16)]
                idxd = dst_v[pl.ds(g * 16, 16)]
                acc = jnp.zeros((16,), F32)
                for c in range(4):
                    xa = plsc.load_gather(xl_v.at[c], [idxs])
                    xb = plsc.load_gather(xr_v.at[c], [idxd])
                    m = xa + xb + el_v[c, pl.ds(g * 16, 16)]
                    m = jnp.maximum(m, 0.2 * m)
                    acc = acc + atts[c] * m
                s_v[pl.ds(g * 16, 16)] = acc
                return c1

            lax.fori_loop(0, CH // 16, body, 0)
            pltpu.sync_copy(s_v, sp_hbm.at[cgi, pl.ds(r0, CH)])
            return c0

        lax.fori_loop(0, nchunk, chunk, 0)

    return phase_a


def _tc_reduce_body(sp_ref, s_ref, g_ref):
    blk = sp_ref[...]
    ssum = blk.sum(axis=0)
    s_ref[...] = ssum
    bm = jnp.max(ssum)

    @pl.when(pl.program_id(0) == 0)
    def _():
        g_ref[...] = jnp.full((1, 128), -1e30, F32)

    g_ref[...] = jnp.maximum(g_ref[...], bm)


def _tc_reduce_s(s_parts3):
    cg = s_parts3.shape[0]
    rows = E_PAD // 128
    rblk = 8
    grid = rows // rblk
    return pl.pallas_call(
        _tc_reduce_body,
        grid=(grid,),
        in_specs=[pl.BlockSpec((cg, rblk, 128), lambda i: (0, i, 0))],
        out_specs=(
            pl.BlockSpec((rblk, 128), lambda i: (i, 0)),
            pl.BlockSpec((1, 128), lambda i: (0, 0)),
        ),
        out_shape=(
            jax.ShapeDtypeStruct((rows, 128), F32),
            jax.ShapeDtypeStruct((1, 128), F32),
        ),
    )(s_parts3)


_B2_PER = E_PAD // 32
_B2_CH = 4064


@functools.partial(
    pl.kernel,
    out_type=(
        jax.ShapeDtypeStruct((E_PAD,), F32),
        jax.ShapeDtypeStruct((32, N_PAD), F32),
    ),
    mesh=_MESH,
    compiler_params=_SC_PARAMS,
    scratch_types=[
        pltpu.VMEM((N_PAD,), F32),
        pltpu.VMEM((_B2_CH,), F32),
        pltpu.VMEM((_B2_CH,), I32),
        pltpu.VMEM((16,), F32),
    ],
)
def _sc_phase_b2(s_hbm, g_hbm, dst_hbm, ex_hbm, dp_hbm, den_v, s_v, dst_v,
                 g_v):
    wid = _wid()
    e0 = wid * _B2_PER
    pltpu.sync_copy(g_hbm.at[0, pl.ds(0, 16)], g_v)
    gv = g_v[...]
    _zero_1d(den_v, N_PAD)

    def chunk(k, c0):
        r0 = e0 + k * _B2_CH
        pltpu.sync_copy(s_hbm.at[pl.ds(r0, _B2_CH)], s_v)
        pltpu.sync_copy(dst_hbm.at[pl.ds(r0, _B2_CH)], dst_v)

        def body(g, c1):
            sv = s_v[pl.ds(g * 16, 16)]
            ex = jnp.exp(sv - gv)
            s_v[pl.ds(g * 16, 16)] = ex
            idx = dst_v[pl.ds(g * 16, 16)]
            plsc.addupdate_scatter(den_v, [idx], ex)
            return c1

        lax.fori_loop(0, _B2_CH // 16, body, 0)
        pltpu.sync_copy(s_v, ex_hbm.at[pl.ds(r0, _B2_CH)])
        return c0

    lax.fori_loop(0, _B2_PER // _B2_CH, chunk, 0)
    pltpu.sync_copy(den_v, dp_hbm.at[wid])


def _tc_rden_body(dp_ref, out_ref):
    out_ref[...] = 1.0 / (dp_ref[...].sum(axis=0) + 1e-16)


def _tc_rden(dp3):
    return pl.pallas_call(
        _tc_rden_body,
        out_shape=jax.ShapeDtypeStruct((N_PAD // 128, 128), F32),
    )(dp3)


def _make_sc_phase_c(out_dim, cg, es):
    el_per = E_PAD // es
    nchunk = el_per // CH

    @functools.partial(
        pl.kernel,
        out_type=jax.ShapeDtypeStruct((es, out_dim, N_PAD), F32),
        mesh=_MESH,
        compiler_params=_SC_PARAMS,
        scratch_types=[
            pltpu.VMEM((4, N_PAD), F32),
            pltpu.VMEM((4, N_PAD), F32),
            pltpu.VMEM((N_PAD,), F32),
            pltpu.VMEM((CH,), I32),
            pltpu.VMEM((CH,), I32),
            pltpu.VMEM((CH,), F32),
        ],
    )
    def phase_c(xlt_hbm, rden_hbm, ex_hbm, src_hbm, dst_hbm, hp_hbm,
                xl_v, h_v, rden_v, src_v, dst_v, ex_v):
        wid = _wid()
        cgi = wid % cg
        esi = wid // cg
        e0 = esi * el_per
        c4 = cgi * 4
        pltpu.sync_copy(rden_hbm, rden_v)
        pltpu.sync_copy(xlt_hbm.at[pl.ds(c4, 4)], xl_v)

        def zb(g, c):
            z = jnp.zeros((16,), F32)
            for cc in range(4):
                h_v[cc, pl.ds(g * 16, 16)] = z
            return c

        lax.fori_loop(0, N_PAD // 16, zb, 0)

        def chunk(k, c0):
            r0 = e0 + k * CH
            pltpu.sync_copy(src_hbm.at[pl.ds(r0, CH)], src_v)
            pltpu.sync_copy(dst_hbm.at[pl.ds(r0, CH)], dst_v)
            pltpu.sync_copy(ex_hbm.at[pl.ds(r0, CH)], ex_v)

            def body(g, c1):
                idxs = src_v[pl.ds(g * 16, 16)]
                idxd = dst_v[pl.ds(g * 16, 16)]
                alpha = ex_v[pl.ds(g * 16, 16)] * plsc.load_gather(
                    rden_v, [idxd])
                for c in range(4):
                    xv = plsc.load_gather(xl_v.at[c], [idxs])
                    plsc.addupdate_scatter(h_v.at[c], [idxd], alpha * xv)
                return c1

            lax.fori_loop(0, CH // 16, body, 0)
            return c0

        lax.fori_loop(0, nchunk, chunk, 0)
        pltpu.sync_copy(h_v, hp_hbm.at[esi, pl.ds(c4, 4)])

    return phase_c


@functools.partial(
    pl.kernel,
    out_type=jax.ShapeDtypeStruct((OUT_GAT, NUM_GRAPHS), F32),
    mesh=_MESH,
    compiler_params=_SC_PARAMS,
    scratch_types=[
        pltpu.VMEM((N_PAD,), I32),
        pltpu.VMEM((N_PAD,), F32),
        pltpu.VMEM((1024,), F32),
        pltpu.VMEM((16,), F32),
    ],
)
def _pool_sc(hT_hbm, batch_hbm, bbc_hbm, out_hbm, batch_v, col_v, pool_v, b_v):
    wid = _wid()
    pltpu.sync_copy(batch_hbm, batch_v)
    for c in range(4):
        col = wid * 4 + c
        pltpu.sync_copy(hT_hbm.at[col], col_v)
        pltpu.sync_copy(bbc_hbm.at[col], b_v)
        bv = b_v[...]
        _zero_1d(pool_v, 1024)

        def body(i, carry):
            v = col_v[pl.ds(i * 16, 16)] + bv
            idx = batch_v[pl.ds(i * 16, 16)]
            plsc.addupdate_scatter(pool_v, [idx], v)
            return carry

        lax.fori_loop(0, N_PAD // 16, body, 0)
        pltpu.sync_copy(pool_v.at[pl.ds(0, NUM_GRAPHS)], out_hbm.at[col])


def _tc_mlp_body(p_ref, w1_ref, b1_ref, w2_ref, b2_ref, out_ref):
    z = jnp.dot(w1_ref[...], p_ref[...], preferred_element_type=F32)
    z = z + b1_ref[...][:, None]
    z = z * jnp.tanh(jax.nn.softplus(z))
    o = jnp.dot(w2_ref[...], z, preferred_element_type=F32)
    out_ref[...] = o + b2_ref[...][:, None]


def _tc_mlp(pooledT, fcWT, fcb, fc2WT, fc2b):
    return pl.pallas_call(
        _tc_mlp_body,
        out_shape=jax.ShapeDtypeStruct((3, NUM_GRAPHS), F32),
    )(pooledT, fcWT, fcb, fc2WT, fc2b)


_PHASE_A_64 = _make_sc_phase_a(HID, 16, 2)
_PHASE_A_128 = _make_sc_phase_a(OUT_GAT, 32, 1)
_PHASE_C_64 = _make_sc_phase_c(HID, 16, 2)
_PHASE_C_128 = _make_sc_phase_c(OUT_GAT, 32, 1)


def kernel(x, edge_index, edge_attr, batch, params):
    src = edge_index[0]
    dst = edge_index[1]
    loop = jnp.arange(N, dtype=I32)
    padi = jnp.full((N_PAD - N,), PAD_NODE, I32)
    src_pad = jnp.concatenate([src, loop, padi])
    dst_pad = jnp.concatenate([dst, loop, padi])
    batch_pad = jnp.full((N_PAD,), NUM_GRAPHS, I32).at[:N].set(batch)
    x_pad = jnp.zeros((N_PAD, D_IN), F32).at[:N].set(x)

    xT = _sc_transpose_x(x_pad)
    eaT = _sc_transpose_ea(edge_attr)
    sums, cnts = _sc_ea_segsum(eaT, dst)
    ea_loopT = _tc_ea_loop(sums, cnts)
    ea_aT = jnp.concatenate([eaT, ea_loopT], axis=1)

    a2d = jnp.reshape(params['prelu_a'], (1, 1)).astype(F32)
    hT = xT
    for l, p in enumerate(params['layers']):
        out_dim = p['Wl'].shape[1]
        WlT = p['Wl'].T
        WrT = p['Wr'].T
        WeT = p['We'].T
        attbc = jnp.broadcast_to(p['att'][:, None], (out_dim, 16))
        xlT, xrT = _tc_proj(hT, WlT, WrT)
        elT = _tc_el(WeT, ea_aT)
        if out_dim == HID:
            cg, es = 16, 2
            sp = _PHASE_A_64(xlT, xrT, elT, attbc, src_pad, dst_pad)
        else:
            cg, es = 32, 1
            sp = _PHASE_A_128(xlT, xrT, elT, attbc, src_pad, dst_pad)
        s2d, gmax = _tc_reduce_s(sp.reshape(cg, E_PAD // 128, 128))
        s1 = s2d.reshape(E_PAD)
        ex, dparts = _sc_phase_b2(s1, gmax, dst_pad)
        rden2 = _tc_rden(dparts.reshape(32, N_PAD // 128, 128))
        rden = rden2.reshape(N_PAD)
        if out_dim == HID:
            h_parts = _PHASE_C_64(xlT, rden, ex, src_pad, dst_pad)
        else:
            h_parts = _PHASE_C_128(xlT, rden, ex, src_pad, dst_pad)
        if l < NUM_LAYERS - 1:
            scale = p['bn_gamma'] / jnp.sqrt(p['bn_var'] + 1e-5)
            shift = p['bn_beta'] - p['bn_mean'] * scale + p['b'] * scale
            hT = _tc_act(h_parts, scale, shift, a2d, es, True)
        else:
            hT128 = h_parts[0]
            bbc = jnp.broadcast_to(p['b'][:, None], (OUT_GAT, 16))
            pooledT = _pool_sc(hT128, batch_pad, bbc)

    out = _tc_mlp(pooledT, params['fc_W'].T, params['fc_b'],
                  params['fc2_W'].T, params['fc2_b'])
    return out.T


# speedup vs baseline: 6.7267x; 1.0663x over previous
import functools

import jax
import jax.numpy as jnp
from jax import lax
from jax.experimental import pallas as pl
from jax.experimental.pallas import tpu as pltpu
from jax.experimental.pallas import tpu_sc as plsc

N = 10000
E = 640000
D_IN = 48
D_EDGE = 16
HID = 64
OUT_GAT = 128
NUM_LAYERS = 6
NUM_GRAPHS = 512

N_PAD = 10240
E_PAD = E + N_PAD
CH = 4064
PAD_NODE = N

_MESH = plsc.VectorSubcoreMesh(core_axis_name="c", subcore_axis_name="s")
_SC_PARAMS = pltpu.CompilerParams(
    needs_layout_passes=False, use_tc_tiling_on_sc=False)

F32 = jnp.float32
I32 = jnp.int32


def _wid():
    return lax.axis_index("s") * 2 + lax.axis_index("c")


def _zero_1d(ref, n):
    def zb(g, c):
        ref[pl.ds(g * 16, 16)] = jnp.zeros((16,), F32)
        return c
    lax.fori_loop(0, n // 16, zb, 0)


@functools.partial(
    pl.kernel,
    out_type=jax.ShapeDtypeStruct((D_IN, N_PAD), F32),
    mesh=_MESH,
    compiler_params=_SC_PARAMS,
    scratch_types=[
        pltpu.VMEM((320, D_IN), F32),
        pltpu.VMEM((D_IN, 320), F32),
    ],
)
def _sc_transpose_x(x_hbm, xt_hbm, row_v, col_v):
    wid = _wid()
    r0 = wid * 320
    pltpu.sync_copy(x_hbm.at[pl.ds(r0, 320)], row_v)
    lanes = jax.lax.iota(I32, 16)

    def body(g, c):
        ridx = g * 16 + lanes
        for cc in range(D_IN):
            v = plsc.load_gather(row_v, [ridx, jnp.full((16,), cc, I32)])
            col_v[cc, pl.ds(g * 16, 16)] = v
        return c

    lax.fori_loop(0, 20, body, 0)
    pltpu.sync_copy(col_v, xt_hbm.at[:, pl.ds(r0, 320)])


@functools.partial(
    pl.kernel,
    out_type=jax.ShapeDtypeStruct((D_EDGE, E), F32),
    mesh=_MESH,
    compiler_params=_SC_PARAMS,
    scratch_types=[
        pltpu.VMEM((2000, D_EDGE), F32),
        pltpu.VMEM((D_EDGE, 2000), F32),
    ],
)
def _sc_transpose_ea(ea_hbm, eat_hbm, row_v, col_v):
    wid = _wid()
    e0 = wid * 20000
    lanes = jax.lax.iota(I32, 16)

    def chunk(k, c0):
        r0 = e0 + k * 2000
        pltpu.sync_copy(ea_hbm.at[pl.ds(r0, 2000)], row_v)

        def body(g, c):
            ridx = g * 16 + lanes
            for cc in range(D_EDGE):
                v = plsc.load_gather(row_v, [ridx, jnp.full((16,), cc, I32)])
                col_v[cc, pl.ds(g * 16, 16)] = v
            return c

        lax.fori_loop(0, 125, body, 0)
        pltpu.sync_copy(col_v, eat_hbm.at[:, pl.ds(r0, 2000)])
        return c0

    lax.fori_loop(0, 10, chunk, 0)


@functools.partial(
    pl.kernel,
    out_type=(
        jax.ShapeDtypeStruct((2, D_EDGE, N_PAD), F32),
        jax.ShapeDtypeStruct((2, N_PAD), F32),
    ),
    mesh=_MESH,
    compiler_params=_SC_PARAMS,
    scratch_types=[
        pltpu.VMEM((N_PAD,), F32),
        pltpu.VMEM((N_PAD,), F32),
        pltpu.VMEM((2000,), F32),
        pltpu.VMEM((2000,), I32),
    ],
)
def _sc_ea_segsum(eat_hbm, dst_hbm, sum_hbm, cnt_hbm, acc_v, cntacc_v, val_v,
                  dst_v):
    wid = _wid()
    col = wid % 16
    half = wid // 16
    e0 = half * (E // 2)
    _zero_1d(acc_v, N_PAD)
    _zero_1d(cntacc_v, N_PAD)
    ones = jnp.ones((16,), F32)

    def chunk(k, c0):
        r0 = e0 + k * 2000
        pltpu.sync_copy(eat_hbm.at[col, pl.ds(r0, 2000)], val_v)
        pltpu.sync_copy(dst_hbm.at[pl.ds(r0, 2000)], dst_v)

        def body(g, c):
            idx = dst_v[pl.ds(g * 16, 16)]
            v = val_v[pl.ds(g * 16, 16)]
            plsc.addupdate_scatter(acc_v, [idx], v)

            @pl.when(col == 0)
            def _():
                plsc.addupdate_scatter(cntacc_v, [idx], ones)

            return c

        lax.fori_loop(0, 125, body, 0)
        return c0

    lax.fori_loop(0, (E // 2) // 2000, chunk, 0)
    pltpu.sync_copy(acc_v, sum_hbm.at[half, col])

    @pl.when(col == 0)
    def _():
        pltpu.sync_copy(cntacc_v, cnt_hbm.at[half])


def _tc_ea_loop_body(sum_ref, cnt_ref, out_ref):
    s = sum_ref[0] + sum_ref[1]
    c = cnt_ref[0] + cnt_ref[1]
    out_ref[...] = s / jnp.maximum(c, 1.0)[None, :]


def _tc_ea_loop(sums, cnts):
    return pl.pallas_call(
        _tc_ea_loop_body,
        out_shape=jax.ShapeDtypeStruct((D_EDGE, N_PAD), F32),
    )(sums, cnts)


def _tc_proj_body(h_ref, wl_ref, wr_ref, xl_ref, xr_ref):
    h = h_ref[...]
    xl_ref[...] = jnp.dot(wl_ref[...], h, preferred_element_type=F32)
    xr_ref[...] = jnp.dot(wr_ref[...], h, preferred_element_type=F32)


def _tc_proj(hT, WlT, WrT):
    out = WlT.shape[0]
    return pl.pallas_call(
        _tc_proj_body,
        out_shape=(
            jax.ShapeDtypeStruct((out, N_PAD), F32),
            jax.ShapeDtypeStruct((out, N_PAD), F32),
        ),
    )(hT, WlT, WrT)


def _tc_act_body(parts_ref, sc_ref, sh_ref, a_ref, out_ref, *, es, bn):
    h = parts_ref[0]
    for i in range(1, es):
        h = h + parts_ref[i]
    if bn:
        h = h * sc_ref[...][:, None] + sh_ref[...][:, None]
        a = a_ref[0, 0]
        h = jnp.where(h >= 0, h, a * h)
    else:
        h = h + sc_ref[...][:, None]
    mask = lax.broadcasted_iota(I32, h.shape, 1) < N
    out_ref[...] = jnp.where(mask, h, 0.0)


def _tc_act(h_parts, scale, shift, a2d, es, bn):
    out = h_parts.shape[1]
    body = functools.partial(_tc_act_body, es=es, bn=bn)
    return pl.pallas_call(
        body,
        out_shape=jax.ShapeDtypeStruct((out, N_PAD), F32),
    )(h_parts, scale, shift, a2d)


def _tc_el_body(we_ref, ea_ref, out_ref):
    out_ref[...] = jnp.dot(we_ref[...], ea_ref[...], preferred_element_type=F32)


def _tc_el(WeT, ea_aT):
    out = WeT.shape[0]
    blk = 5120
    grid = E_PAD // blk
    return pl.pallas_call(
        _tc_el_body,
        grid=(grid,),
        in_specs=[
            pl.BlockSpec((out, D_EDGE), lambda i: (0, 0)),
            pl.BlockSpec((D_EDGE, blk), lambda i: (0, i)),
        ],
        out_specs=pl.BlockSpec((out, blk), lambda i: (0, i)),
        out_shape=jax.ShapeDtypeStruct((out, E_PAD), F32),
    )(WeT, ea_aT)


def _make_sc_phase_a(out_dim, cg, es):
    el_per = E_PAD // es
    nchunk = el_per // CH

    @functools.partial(
        pl.kernel,
        out_type=jax.ShapeDtypeStruct((cg, E_PAD), F32),
        mesh=_MESH,
        compiler_params=_SC_PARAMS,
        scratch_types=[
            pltpu.VMEM((4, N_PAD), F32),
            pltpu.VMEM((4, N_PAD), F32),
            pltpu.VMEM((4, 16), F32),
            pltpu.VMEM((CH,), I32),
            pltpu.VMEM((CH,), I32),
            pltpu.VMEM((4, CH), F32),
            pltpu.VMEM((CH,), F32),
        ],
    )
    def phase_a(xlt_hbm, xrt_hbm, elt_hbm, attbc_hbm, src_hbm, dst_hbm,
                sp_hbm, xl_v, xr_v, att_v, src_v, dst_v, el_v, s_v):
        wid = _wid()
        cgi = wid % cg
        esi = wid // cg
        e0 = esi * el_per
        c4 = cgi * 4
        pltpu.sync_copy(xlt_hbm.at[pl.ds(c4, 4)], xl_v)
        pltpu.sync_copy(xrt_hbm.at[pl.ds(c4, 4)], xr_v)
        pltpu.sync_copy(attbc_hbm.at[pl.ds(c4, 4)], att_v)
        atts = tuple(att_v[c] for c in range(4))

        def chunk(k, c0):
            r0 = e0 + k * CH
            pltpu.sync_copy(src_hbm.at[pl.ds(r0, CH)], src_v)
            pltpu.sync_copy(dst_hbm.at[pl.ds(r0, CH)], dst_v)
            pltpu.sync_copy(elt_hbm.at[pl.ds(c4, 4), pl.ds(r0, CH)], el_v)

            @plsc.parallel_loop(0, CH // 16, unroll=4)
            def body(g):
                idxs = src_v[pl.ds(g * 16, 16)]
                idxd = dst_v[pl.ds(g * 16, 16)]
                acc = jnp.zeros((16,), F32)
                for c in range(4):
                    xa = plsc.load_gather(xl_v.at[c], [idxs])
                    xb = plsc.load_gather(xr_v.at[c], [idxd])
                    m = xa + xb + el_v[c, pl.ds(g * 16, 16)]
                    m = jnp.maximum(m, 0.2 * m)
                    acc = acc + atts[c] * m
                s_v[pl.ds(g * 16, 16)] = acc
            pltpu.sync_copy(s_v, sp_hbm.at[cgi, pl.ds(r0, CH)])
            return c0

        lax.fori_loop(0, nchunk, chunk, 0)

    return phase_a


def _tc_reduce_body(sp_ref, s_ref, g_ref):
    blk = sp_ref[...]
    ssum = blk.sum(axis=0)
    s_ref[...] = ssum
    bm = jnp.max(ssum)

    @pl.when(pl.program_id(0) == 0)
    def _():
        g_ref[...] = jnp.full((1, 128), -1e30, F32)

    g_ref[...] = jnp.maximum(g_ref[...], bm)


def _tc_reduce_s(s_parts3):
    cg = s_parts3.shape[0]
    rows = E_PAD // 128
    rblk = 8
    grid = rows // rblk
    return pl.pallas_call(
        _tc_reduce_body,
        grid=(grid,),
        in_specs=[pl.BlockSpec((cg, rblk, 128), lambda i: (0, i, 0))],
        out_specs=(
            pl.BlockSpec((rblk, 128), lambda i: (i, 0)),
            pl.BlockSpec((1, 128), lambda i: (0, 0)),
        ),
        out_shape=(
            jax.ShapeDtypeStruct((rows, 128), F32),
            jax.ShapeDtypeStruct((1, 128), F32),
        ),
    )(s_parts3)


_B2_PER = E_PAD // 32
_B2_CH = 4064


@functools.partial(
    pl.kernel,
    out_type=(
        jax.ShapeDtypeStruct((E_PAD,), F32),
        jax.ShapeDtypeStruct((32, N_PAD), F32),
    ),
    mesh=_MESH,
    compiler_params=_SC_PARAMS,
    scratch_types=[
        pltpu.VMEM((N_PAD,), F32),
        pltpu.VMEM((_B2_CH,), F32),
        pltpu.VMEM((_B2_CH,), I32),
        pltpu.VMEM((16,), F32),
    ],
)
def _sc_phase_b2(s_hbm, g_hbm, dst_hbm, ex_hbm, dp_hbm, den_v, s_v, dst_v,
                 g_v):
    wid = _wid()
    e0 = wid * _B2_PER
    pltpu.sync_copy(g_hbm.at[0, pl.ds(0, 16)], g_v)
    gv = g_v[...]
    _zero_1d(den_v, N_PAD)

    def chunk(k, c0):
        r0 = e0 + k * _B2_CH
        pltpu.sync_copy(s_hbm.at[pl.ds(r0, _B2_CH)], s_v)
        pltpu.sync_copy(dst_hbm.at[pl.ds(r0, _B2_CH)], dst_v)

        def body(g4, c1):
            for u in range(4):
                g = g4 * 4 + u
                sv = s_v[pl.ds(g * 16, 16)]
                ex = jnp.exp(sv - gv)
                s_v[pl.ds(g * 16, 16)] = ex
                idx = dst_v[pl.ds(g * 16, 16)]
                plsc.addupdate_scatter(den_v, [idx], ex)
            return c1

        lax.fori_loop(0, _B2_CH // 64, body, 0)
        pltpu.sync_copy(s_v, ex_hbm.at[pl.ds(r0, _B2_CH)])
        return c0

    lax.fori_loop(0, _B2_PER // _B2_CH, chunk, 0)
    pltpu.sync_copy(den_v, dp_hbm.at[wid])


def _tc_rden_body(dp_ref, out_ref):
    out_ref[...] = 1.0 / (dp_ref[...].sum(axis=0) + 1e-16)


def _tc_rden(dp3):
    return pl.pallas_call(
        _tc_rden_body,
        out_shape=jax.ShapeDtypeStruct((N_PAD // 128, 128), F32),
    )(dp3)


def _make_sc_phase_c(out_dim, cg, es):
    el_per = E_PAD // es
    nchunk = el_per // CH

    @functools.partial(
        pl.kernel,
        out_type=jax.ShapeDtypeStruct((es, out_dim, N_PAD), F32),
        mesh=_MESH,
        compiler_params=_SC_PARAMS,
        scratch_types=[
            pltpu.VMEM((4, N_PAD), F32),
            pltpu.VMEM((4, N_PAD), F32),
            pltpu.VMEM((N_PAD,), F32),
            pltpu.VMEM((CH,), I32),
            pltpu.VMEM((CH,), I32),
            pltpu.VMEM((CH,), F32),
        ],
    )
    def phase_c(xlt_hbm, rden_hbm, ex_hbm, src_hbm, dst_hbm, hp_hbm,
                xl_v, h_v, rden_v, src_v, dst_v, ex_v):
        wid = _wid()
        cgi = wid % cg
        esi = wid // cg
        e0 = esi * el_per
        c4 = cgi * 4
        pltpu.sync_copy(rden_hbm, rden_v)
        pltpu.sync_copy(xlt_hbm.at[pl.ds(c4, 4)], xl_v)

        def zb(g, c):
            z = jnp.zeros((16,), F32)
            for cc in range(4):
                h_v[cc, pl.ds(g * 16, 16)] = z
            return c

        lax.fori_loop(0, N_PAD // 16, zb, 0)

        def chunk(k, c0):
            r0 = e0 + k * CH
            pltpu.sync_copy(src_hbm.at[pl.ds(r0, CH)], src_v)
            pltpu.sync_copy(dst_hbm.at[pl.ds(r0, CH)], dst_v)
            pltpu.sync_copy(ex_hbm.at[pl.ds(r0, CH)], ex_v)

            def body(g4, c1):
                for u in range(4):
                    g = g4 * 4 + u
                    idxs = src_v[pl.ds(g * 16, 16)]
                    idxd = dst_v[pl.ds(g * 16, 16)]
                    alpha = ex_v[pl.ds(g * 16, 16)] * plsc.load_gather(
                        rden_v, [idxd])
                    for c in range(4):
                        xv = plsc.load_gather(xl_v.at[c], [idxs])
                        plsc.addupdate_scatter(h_v.at[c], [idxd], alpha * xv)
                return c1

            lax.fori_loop(0, CH // 64, body, 0)
            return c0

        lax.fori_loop(0, nchunk, chunk, 0)
        pltpu.sync_copy(h_v, hp_hbm.at[esi, pl.ds(c4, 4)])

    return phase_c


@functools.partial(
    pl.kernel,
    out_type=jax.ShapeDtypeStruct((OUT_GAT, NUM_GRAPHS), F32),
    mesh=_MESH,
    compiler_params=_SC_PARAMS,
    scratch_types=[
        pltpu.VMEM((N_PAD,), I32),
        pltpu.VMEM((N_PAD,), F32),
        pltpu.VMEM((1024,), F32),
        pltpu.VMEM((16,), F32),
    ],
)
def _pool_sc(hT_hbm, batch_hbm, bbc_hbm, out_hbm, batch_v, col_v, pool_v, b_v):
    wid = _wid()
    pltpu.sync_copy(batch_hbm, batch_v)
    for c in range(4):
        col = wid * 4 + c
        pltpu.sync_copy(hT_hbm.at[col], col_v)
        pltpu.sync_copy(bbc_hbm.at[col], b_v)
        bv = b_v[...]
        _zero_1d(pool_v, 1024)

        def body(i, carry):
            v = col_v[pl.ds(i * 16, 16)] + bv
            idx = batch_v[pl.ds(i * 16, 16)]
            plsc.addupdate_scatter(pool_v, [idx], v)
            return carry

        lax.fori_loop(0, N_PAD // 16, body, 0)
        pltpu.sync_copy(pool_v.at[pl.ds(0, NUM_GRAPHS)], out_hbm.at[col])


def _tc_mlp_body(p_ref, w1_ref, b1_ref, w2_ref, b2_ref, out_ref):
    z = jnp.dot(w1_ref[...], p_ref[...], preferred_element_type=F32)
    z = z + b1_ref[...][:, None]
    z = z * jnp.tanh(jax.nn.softplus(z))
    o = jnp.dot(w2_ref[...], z, preferred_element_type=F32)
    out_ref[...] = o + b2_ref[...][:, None]


def _tc_mlp(pooledT, fcWT, fcb, fc2WT, fc2b):
    return pl.pallas_call(
        _tc_mlp_body,
        out_shape=jax.ShapeDtypeStruct((3, NUM_GRAPHS), F32),
    )(pooledT, fcWT, fcb, fc2WT, fc2b)


_PHASE_A_64 = _make_sc_phase_a(HID, 16, 2)
_PHASE_A_128 = _make_sc_phase_a(OUT_GAT, 32, 1)
_PHASE_C_64 = _make_sc_phase_c(HID, 16, 2)
_PHASE_C_128 = _make_sc_phase_c(OUT_GAT, 32, 1)


def kernel(x, edge_index, edge_attr, batch, params):
    src = edge_index[0]
    dst = edge_index[1]
    loop = jnp.arange(N, dtype=I32)
    padi = jnp.full((N_PAD - N,), PAD_NODE, I32)
    src_pad = jnp.concatenate([src, loop, padi])
    dst_pad = jnp.concatenate([dst, loop, padi])
    batch_pad = jnp.full((N_PAD,), NUM_GRAPHS, I32).at[:N].set(batch)
    x_pad = jnp.zeros((N_PAD, D_IN), F32).at[:N].set(x)

    xT = _sc_transpose_x(x_pad)
    eaT = _sc_transpose_ea(edge_attr)
    sums, cnts = _sc_ea_segsum(eaT, dst)
    ea_loopT = _tc_ea_loop(sums, cnts)
    ea_aT = jnp.concatenate([eaT, ea_loopT], axis=1)

    a2d = jnp.reshape(params['prelu_a'], (1, 1)).astype(F32)
    hT = xT
    for l, p in enumerate(params['layers']):
        out_dim = p['Wl'].shape[1]
        WlT = p['Wl'].T
        WrT = p['Wr'].T
        WeT = p['We'].T
        attbc = jnp.broadcast_to(p['att'][:, None], (out_dim, 16))
        xlT, xrT = _tc_proj(hT, WlT, WrT)
        elT = _tc_el(WeT, ea_aT)
        if out_dim == HID:
            cg, es = 16, 2
            sp = _PHASE_A_64(xlT, xrT, elT, attbc, src_pad, dst_pad)
        else:
            cg, es = 32, 1
            sp = _PHASE_A_128(xlT, xrT, elT, attbc, src_pad, dst_pad)
        s2d, gmax = _tc_reduce_s(sp.reshape(cg, E_PAD // 128, 128))
        s1 = s2d.reshape(E_PAD)
        ex, dparts = _sc_phase_b2(s1, gmax, dst_pad)
        rden2 = _tc_rden(dparts.reshape(32, N_PAD // 128, 128))
        rden = rden2.reshape(N_PAD)
        if out_dim == HID:
            h_parts = _PHASE_C_64(xlT, rden, ex, src_pad, dst_pad)
        else:
            h_parts = _PHASE_C_128(xlT, rden, ex, src_pad, dst_pad)
        if l < NUM_LAYERS - 1:
            scale = p['bn_gamma'] / jnp.sqrt(p['bn_var'] + 1e-5)
            shift = p['bn_beta'] - p['bn_mean'] * scale + p['b'] * scale
            hT = _tc_act(h_parts, scale, shift, a2d, es, True)
        else:
            hT128 = h_parts[0]
            bbc = jnp.broadcast_to(p['b'][:, None], (OUT_GAT, 16))
            pooledT = _pool_sc(hT128, batch_pad, bbc)

    out = _tc_mlp(pooledT, params['fc_W'].T, params['fc_b'],
                  params['fc2_W'].T, params['fc2_b'])
    return out.T


# speedup vs baseline: 8.4676x; 1.2588x over previous
import functools

import jax
import jax.numpy as jnp
from jax import lax
from jax.experimental import pallas as pl
from jax.experimental.pallas import tpu as pltpu
from jax.experimental.pallas import tpu_sc as plsc

N = 10000
E = 640000
D_IN = 48
D_EDGE = 16
HID = 64
OUT_GAT = 128
NUM_LAYERS = 6
NUM_GRAPHS = 512

N_PAD = 10240
E_PAD = E + N_PAD
CH = 4064
PAD_NODE = N

_MESH = plsc.VectorSubcoreMesh(core_axis_name="c", subcore_axis_name="s")
_SC_PARAMS = pltpu.CompilerParams(
    needs_layout_passes=False, use_tc_tiling_on_sc=False)

F32 = jnp.float32
I32 = jnp.int32


def _wid():
    return lax.axis_index("s") * 2 + lax.axis_index("c")


def _zero_1d(ref, n):
    def zb(g, c):
        ref[pl.ds(g * 16, 16)] = jnp.zeros((16,), F32)
        return c
    lax.fori_loop(0, n // 16, zb, 0)


@functools.partial(
    pl.kernel,
    out_type=jax.ShapeDtypeStruct((D_IN, N_PAD), F32),
    mesh=_MESH,
    compiler_params=_SC_PARAMS,
    scratch_types=[
        pltpu.VMEM((320, D_IN), F32),
        pltpu.VMEM((D_IN, 320), F32),
    ],
)
def _sc_transpose_x(x_hbm, xt_hbm, row_v, col_v):
    wid = _wid()
    r0 = wid * 320
    pltpu.sync_copy(x_hbm.at[pl.ds(r0, 320)], row_v)
    lanes = jax.lax.iota(I32, 16)

    def body(g, c):
        ridx = g * 16 + lanes
        for cc in range(D_IN):
            v = plsc.load_gather(row_v, [ridx, jnp.full((16,), cc, I32)])
            col_v[cc, pl.ds(g * 16, 16)] = v
        return c

    lax.fori_loop(0, 20, body, 0)
    pltpu.sync_copy(col_v, xt_hbm.at[:, pl.ds(r0, 320)])


@functools.partial(
    pl.kernel,
    out_type=jax.ShapeDtypeStruct((D_EDGE, E), F32),
    mesh=_MESH,
    compiler_params=_SC_PARAMS,
    scratch_types=[
        pltpu.VMEM((2000, D_EDGE), F32),
        pltpu.VMEM((D_EDGE, 2000), F32),
    ],
)
def _sc_transpose_ea(ea_hbm, eat_hbm, row_v, col_v):
    wid = _wid()
    e0 = wid * 20000
    lanes = jax.lax.iota(I32, 16)

    def chunk(k, c0):
        r0 = e0 + k * 2000
        pltpu.sync_copy(ea_hbm.at[pl.ds(r0, 2000)], row_v)

        def body(g, c):
            ridx = g * 16 + lanes
            for cc in range(D_EDGE):
                v = plsc.load_gather(row_v, [ridx, jnp.full((16,), cc, I32)])
                col_v[cc, pl.ds(g * 16, 16)] = v
            return c

        lax.fori_loop(0, 125, body, 0)
        pltpu.sync_copy(col_v, eat_hbm.at[:, pl.ds(r0, 2000)])
        return c0

    lax.fori_loop(0, 10, chunk, 0)


@functools.partial(
    pl.kernel,
    out_type=(
        jax.ShapeDtypeStruct((2, D_EDGE, N_PAD), F32),
        jax.ShapeDtypeStruct((2, N_PAD), F32),
    ),
    mesh=_MESH,
    compiler_params=_SC_PARAMS,
    scratch_types=[
        pltpu.VMEM((N_PAD,), F32),
        pltpu.VMEM((N_PAD,), F32),
        pltpu.VMEM((2000,), F32),
        pltpu.VMEM((2000,), I32),
    ],
)
def _sc_ea_segsum(eat_hbm, dst_hbm, sum_hbm, cnt_hbm, acc_v, cntacc_v, val_v,
                  dst_v):
    wid = _wid()
    col = wid % 16
    half = wid // 16
    e0 = half * (E // 2)
    _zero_1d(acc_v, N_PAD)
    _zero_1d(cntacc_v, N_PAD)
    ones = jnp.ones((16,), F32)

    def chunk(k, c0):
        r0 = e0 + k * 2000
        pltpu.sync_copy(eat_hbm.at[col, pl.ds(r0, 2000)], val_v)
        pltpu.sync_copy(dst_hbm.at[pl.ds(r0, 2000)], dst_v)

        def body(g, c):
            idx = dst_v[pl.ds(g * 16, 16)]
            v = val_v[pl.ds(g * 16, 16)]
            plsc.addupdate_scatter(acc_v, [idx], v)

            @pl.when(col == 0)
            def _():
                plsc.addupdate_scatter(cntacc_v, [idx], ones)

            return c

        lax.fori_loop(0, 125, body, 0)
        return c0

    lax.fori_loop(0, (E // 2) // 2000, chunk, 0)
    pltpu.sync_copy(acc_v, sum_hbm.at[half, col])

    @pl.when(col == 0)
    def _():
        pltpu.sync_copy(cntacc_v, cnt_hbm.at[half])


def _tc_ea_loop_body(sum_ref, cnt_ref, out_ref):
    s = sum_ref[0] + sum_ref[1]
    c = cnt_ref[0] + cnt_ref[1]
    out_ref[...] = s / jnp.maximum(c, 1.0)[None, :]


def _tc_ea_loop(sums, cnts):
    return pl.pallas_call(
        _tc_ea_loop_body,
        out_shape=jax.ShapeDtypeStruct((D_EDGE, N_PAD), F32),
    )(sums, cnts)


def _tc_proj_body(h_ref, wl_ref, wr_ref, xl_ref, xr_ref):
    h = h_ref[...]
    xl_ref[...] = jnp.dot(wl_ref[...], h, preferred_element_type=F32)
    xr_ref[...] = jnp.dot(wr_ref[...], h, preferred_element_type=F32)


def _tc_proj(hT, WlT, WrT):
    out = WlT.shape[0]
    return pl.pallas_call(
        _tc_proj_body,
        out_shape=(
            jax.ShapeDtypeStruct((out, N_PAD), F32),
            jax.ShapeDtypeStruct((out, N_PAD), F32),
        ),
    )(hT, WlT, WrT)


def _tc_act_body(parts_ref, sc_ref, sh_ref, a_ref, out_ref, *, es, bn):
    h = parts_ref[0]
    for i in range(1, es):
        h = h + parts_ref[i]
    if bn:
        h = h * sc_ref[...][:, None] + sh_ref[...][:, None]
        a = a_ref[0, 0]
        h = jnp.where(h >= 0, h, a * h)
    else:
        h = h + sc_ref[...][:, None]
    mask = lax.broadcasted_iota(I32, h.shape, 1) < N
    out_ref[...] = jnp.where(mask, h, 0.0)


def _tc_act(h_parts, scale, shift, a2d, es, bn):
    out = h_parts.shape[1]
    body = functools.partial(_tc_act_body, es=es, bn=bn)
    return pl.pallas_call(
        body,
        out_shape=jax.ShapeDtypeStruct((out, N_PAD), F32),
    )(h_parts, scale, shift, a2d)


def _tc_el_body(we_ref, ea_ref, out_ref):
    out_ref[...] = jnp.dot(we_ref[...], ea_ref[...], preferred_element_type=F32)


def _tc_el(WeT, ea_aT):
    out = WeT.shape[0]
    blk = 5120
    grid = E_PAD // blk
    return pl.pallas_call(
        _tc_el_body,
        grid=(grid,),
        in_specs=[
            pl.BlockSpec((out, D_EDGE), lambda i: (0, 0)),
            pl.BlockSpec((D_EDGE, blk), lambda i: (0, i)),
        ],
        out_specs=pl.BlockSpec((out, blk), lambda i: (0, i)),
        out_shape=jax.ShapeDtypeStruct((out, E_PAD), F32),
    )(WeT, ea_aT)


def _make_sc_phase_a(out_dim, cg, es):
    el_per = E_PAD // es
    CHA = 2032
    nchunk = el_per // CHA

    @functools.partial(
        pl.kernel,
        out_type=jax.ShapeDtypeStruct((cg, E_PAD), F32),
        mesh=_MESH,
        compiler_params=_SC_PARAMS,
        scratch_types=[
            pltpu.VMEM((4, N_PAD), F32),
            pltpu.VMEM((4, N_PAD), F32),
            pltpu.VMEM((4, 16), F32),
            pltpu.VMEM((2, CHA), I32),
            pltpu.VMEM((2, CHA), I32),
            pltpu.VMEM((2, 4, CHA), F32),
            pltpu.VMEM((CHA,), F32),
            pltpu.SemaphoreType.DMA,
            pltpu.SemaphoreType.DMA,
        ],
    )
    def phase_a(xlt_hbm, xrt_hbm, elt_hbm, attbc_hbm, src_hbm, dst_hbm,
                sp_hbm, xl_v, xr_v, att_v, src_v, dst_v, el_v, s_v,
                sem0, sem1):
        wid = _wid()
        cgi = wid % cg
        esi = wid // cg
        e0 = esi * el_per
        c4 = cgi * 4
        pltpu.sync_copy(xlt_hbm.at[pl.ds(c4, 4)], xl_v)
        pltpu.sync_copy(xrt_hbm.at[pl.ds(c4, 4)], xr_v)
        pltpu.sync_copy(attbc_hbm.at[pl.ds(c4, 4)], att_v)
        atts = tuple(att_v[c] for c in range(4))
        sems = (sem0, sem1)

        def copies(k, b):
            r0 = e0 + k * CHA
            return (
                (src_hbm.at[pl.ds(r0, CHA)], src_v.at[b]),
                (dst_hbm.at[pl.ds(r0, CHA)], dst_v.at[b]),
                (elt_hbm.at[pl.ds(c4, 4), pl.ds(r0, CHA)], el_v.at[b]),
            )

        def start_in(k, b):
            for s, d in copies(k, b):
                pltpu.async_copy(s, d, sems[b])

        def wait_in(k, b):
            for s, d in copies(k, b):
                pltpu.make_async_copy(s, d, sems[b]).wait()

        def compute(k, b):
            r0 = e0 + k * CHA

            @plsc.parallel_loop(0, CHA // 16, unroll=4)
            def body(g):
                idxs = src_v[b, pl.ds(g * 16, 16)]
                idxd = dst_v[b, pl.ds(g * 16, 16)]
                acc = jnp.zeros((16,), F32)
                for c in range(4):
                    xa = plsc.load_gather(xl_v.at[c], [idxs])
                    xb = plsc.load_gather(xr_v.at[c], [idxd])
                    m = xa + xb + el_v[b, c, pl.ds(g * 16, 16)]
                    m = jnp.maximum(m, 0.2 * m)
                    acc = acc + atts[c] * m
                s_v[pl.ds(g * 16, 16)] = acc
            pltpu.sync_copy(s_v, sp_hbm.at[cgi, pl.ds(r0, CHA)])

        last = nchunk - 1
        start_in(0, 0)

        def pair(j, c0):
            k0 = 2 * j
            start_in(jnp.minimum(k0 + 1, last), 1)
            wait_in(k0, 0)
            compute(k0, 0)
            start_in(jnp.minimum(k0 + 2, last), 0)
            wait_in(jnp.minimum(k0 + 1, last), 1)
            compute(k0 + 1, 1)
            return c0

        lax.fori_loop(0, nchunk // 2, pair, 0)
        wait_in(last, 0)

    return phase_a


def _tc_reduce_body(sp_ref, s_ref, g_ref):
    blk = sp_ref[...]
    ssum = blk.sum(axis=0)
    s_ref[...] = ssum
    bm = jnp.max(ssum)

    @pl.when(pl.program_id(0) == 0)
    def _():
        g_ref[...] = jnp.full((1, 128), -1e30, F32)

    g_ref[...] = jnp.maximum(g_ref[...], bm)


def _tc_reduce_s(s_parts3):
    cg = s_parts3.shape[0]
    rows = E_PAD // 128
    rblk = 8
    grid = rows // rblk
    return pl.pallas_call(
        _tc_reduce_body,
        grid=(grid,),
        in_specs=[pl.BlockSpec((cg, rblk, 128), lambda i: (0, i, 0))],
        out_specs=(
            pl.BlockSpec((rblk, 128), lambda i: (i, 0)),
            pl.BlockSpec((1, 128), lambda i: (0, 0)),
        ),
        out_shape=(
            jax.ShapeDtypeStruct((rows, 128), F32),
            jax.ShapeDtypeStruct((1, 128), F32),
        ),
    )(s_parts3)


_B2_PER = E_PAD // 32
_B2_CH = 4064


@functools.partial(
    pl.kernel,
    out_type=(
        jax.ShapeDtypeStruct((E_PAD,), F32),
        jax.ShapeDtypeStruct((32, N_PAD), F32),
    ),
    mesh=_MESH,
    compiler_params=_SC_PARAMS,
    scratch_types=[
        pltpu.VMEM((N_PAD,), F32),
        pltpu.VMEM((_B2_CH,), F32),
        pltpu.VMEM((_B2_CH,), I32),
        pltpu.VMEM((16,), F32),
    ],
)
def _sc_phase_b2(s_hbm, g_hbm, dst_hbm, ex_hbm, dp_hbm, den_v, s_v, dst_v,
                 g_v):
    wid = _wid()
    e0 = wid * _B2_PER
    pltpu.sync_copy(g_hbm.at[0, pl.ds(0, 16)], g_v)
    gv = g_v[...]
    _zero_1d(den_v, N_PAD)

    def chunk(k, c0):
        r0 = e0 + k * _B2_CH
        pltpu.sync_copy(s_hbm.at[pl.ds(r0, _B2_CH)], s_v)
        pltpu.sync_copy(dst_hbm.at[pl.ds(r0, _B2_CH)], dst_v)

        def body(g4, c1):
            for u in range(4):
                g = g4 * 4 + u
                sv = s_v[pl.ds(g * 16, 16)]
                ex = jnp.exp(sv - gv)
                s_v[pl.ds(g * 16, 16)] = ex
                idx = dst_v[pl.ds(g * 16, 16)]
                plsc.addupdate_scatter(den_v, [idx], ex)
            return c1

        lax.fori_loop(0, _B2_CH // 64, body, 0)
        pltpu.sync_copy(s_v, ex_hbm.at[pl.ds(r0, _B2_CH)])
        return c0

    lax.fori_loop(0, _B2_PER // _B2_CH, chunk, 0)
    pltpu.sync_copy(den_v, dp_hbm.at[wid])


def _tc_rden_body(dp_ref, out_ref):
    out_ref[...] = 1.0 / (dp_ref[...].sum(axis=0) + 1e-16)


def _tc_rden(dp3):
    return pl.pallas_call(
        _tc_rden_body,
        out_shape=jax.ShapeDtypeStruct((N_PAD // 128, 128), F32),
    )(dp3)


def _make_sc_phase_c(out_dim, cg, es):
    el_per = E_PAD // es
    nchunk = el_per // CH

    @functools.partial(
        pl.kernel,
        out_type=jax.ShapeDtypeStruct((es, out_dim, N_PAD), F32),
        mesh=_MESH,
        compiler_params=_SC_PARAMS,
        scratch_types=[
            pltpu.VMEM((4, N_PAD), F32),
            pltpu.VMEM((4, N_PAD), F32),
            pltpu.VMEM((N_PAD,), F32),
            pltpu.VMEM((2, CH), I32),
            pltpu.VMEM((2, CH), I32),
            pltpu.VMEM((2, CH), F32),
            pltpu.SemaphoreType.DMA,
            pltpu.SemaphoreType.DMA,
        ],
    )
    def phase_c(xlt_hbm, rden_hbm, ex_hbm, src_hbm, dst_hbm, hp_hbm,
                xl_v, h_v, rden_v, src_v, dst_v, ex_v, sem0, sem1):
        wid = _wid()
        cgi = wid % cg
        esi = wid // cg
        e0 = esi * el_per
        c4 = cgi * 4
        sems = (sem0, sem1)
        pltpu.sync_copy(rden_hbm, rden_v)
        pltpu.sync_copy(xlt_hbm.at[pl.ds(c4, 4)], xl_v)

        def zb(g, c):
            z = jnp.zeros((16,), F32)
            for cc in range(4):
                h_v[cc, pl.ds(g * 16, 16)] = z
            return c

        lax.fori_loop(0, N_PAD // 16, zb, 0)

        def copies(k, b):
            r0 = e0 + k * CH
            return (
                (src_hbm.at[pl.ds(r0, CH)], src_v.at[b]),
                (dst_hbm.at[pl.ds(r0, CH)], dst_v.at[b]),
                (ex_hbm.at[pl.ds(r0, CH)], ex_v.at[b]),
            )

        def start_in(k, b):
            for s, d in copies(k, b):
                pltpu.async_copy(s, d, sems[b])

        def wait_in(k, b):
            for s, d in copies(k, b):
                pltpu.make_async_copy(s, d, sems[b]).wait()

        def compute(k, b):
            def body(g4, c1):
                for u in range(4):
                    g = g4 * 4 + u
                    idxs = src_v[b, pl.ds(g * 16, 16)]
                    idxd = dst_v[b, pl.ds(g * 16, 16)]
                    alpha = ex_v[b, pl.ds(g * 16, 16)] * plsc.load_gather(
                        rden_v, [idxd])
                    for c in range(4):
                        xv = plsc.load_gather(xl_v.at[c], [idxs])
                        plsc.addupdate_scatter(h_v.at[c], [idxd], alpha * xv)
                return c1

            lax.fori_loop(0, CH // 64, body, 0)

        last = nchunk - 1
        start_in(0, 0)

        def pair(j, c0):
            k0 = 2 * j
            start_in(jnp.minimum(k0 + 1, last), 1)
            wait_in(k0, 0)
            compute(k0, 0)
            start_in(jnp.minimum(k0 + 2, last), 0)
            wait_in(jnp.minimum(k0 + 1, last), 1)
            compute(k0 + 1, 1)
            return c0

        lax.fori_loop(0, nchunk // 2, pair, 0)
        wait_in(last, 0)
        pltpu.sync_copy(h_v, hp_hbm.at[esi, pl.ds(c4, 4)])

    return phase_c


@functools.partial(
    pl.kernel,
    out_type=jax.ShapeDtypeStruct((OUT_GAT, NUM_GRAPHS), F32),
    mesh=_MESH,
    compiler_params=_SC_PARAMS,
    scratch_types=[
        pltpu.VMEM((N_PAD,), I32),
        pltpu.VMEM((N_PAD,), F32),
        pltpu.VMEM((1024,), F32),
        pltpu.VMEM((16,), F32),
    ],
)
def _pool_sc(hT_hbm, batch_hbm, bbc_hbm, out_hbm, batch_v, col_v, pool_v, b_v):
    wid = _wid()
    pltpu.sync_copy(batch_hbm, batch_v)
    for c in range(4):
        col = wid * 4 + c
        pltpu.sync_copy(hT_hbm.at[col], col_v)
        pltpu.sync_copy(bbc_hbm.at[col], b_v)
        bv = b_v[...]
        _zero_1d(pool_v, 1024)

        def body(i, carry):
            v = col_v[pl.ds(i * 16, 16)] + bv
            idx = batch_v[pl.ds(i * 16, 16)]
            plsc.addupdate_scatter(pool_v, [idx], v)
            return carry

        lax.fori_loop(0, N_PAD // 16, body, 0)
        pltpu.sync_copy(pool_v.at[pl.ds(0, NUM_GRAPHS)], out_hbm.at[col])


def _tc_mlp_body(p_ref, w1_ref, b1_ref, w2_ref, b2_ref, out_ref):
    z = jnp.dot(w1_ref[...], p_ref[...], preferred_element_type=F32)
    z = z + b1_ref[...][:, None]
    z = z * jnp.tanh(jax.nn.softplus(z))
    o = jnp.dot(w2_ref[...], z, preferred_element_type=F32)
    out_ref[...] = o + b2_ref[...][:, None]


def _tc_mlp(pooledT, fcWT, fcb, fc2WT, fc2b):
    return pl.pallas_call(
        _tc_mlp_body,
        out_shape=jax.ShapeDtypeStruct((3, NUM_GRAPHS), F32),
    )(pooledT, fcWT, fcb, fc2WT, fc2b)


_PHASE_A_64 = _make_sc_phase_a(HID, 16, 2)
_PHASE_A_128 = _make_sc_phase_a(OUT_GAT, 32, 1)
_PHASE_C_64 = _make_sc_phase_c(HID, 16, 2)
_PHASE_C_128 = _make_sc_phase_c(OUT_GAT, 32, 1)


def kernel(x, edge_index, edge_attr, batch, params):
    src = edge_index[0]
    dst = edge_index[1]
    loop = jnp.arange(N, dtype=I32)
    padi = jnp.full((N_PAD - N,), PAD_NODE, I32)
    src_pad = jnp.concatenate([src, loop, padi])
    dst_pad = jnp.concatenate([dst, loop, padi])
    batch_pad = jnp.full((N_PAD,), NUM_GRAPHS, I32).at[:N].set(batch)
    x_pad = jnp.zeros((N_PAD, D_IN), F32).at[:N].set(x)

    xT = _sc_transpose_x(x_pad)
    eaT = _sc_transpose_ea(edge_attr)
    sums, cnts = _sc_ea_segsum(eaT, dst)
    ea_loopT = _tc_ea_loop(sums, cnts)
    ea_aT = jnp.concatenate([eaT, ea_loopT], axis=1)

    a2d = jnp.reshape(params['prelu_a'], (1, 1)).astype(F32)
    hT = xT
    for l, p in enumerate(params['layers']):
        out_dim = p['Wl'].shape[1]
        WlT = p['Wl'].T
        WrT = p['Wr'].T
        WeT = p['We'].T
        attbc = jnp.broadcast_to(p['att'][:, None], (out_dim, 16))
        xlT, xrT = _tc_proj(hT, WlT, WrT)
        elT = _tc_el(WeT, ea_aT)
        if out_dim == HID:
            cg, es = 16, 2
            sp = _PHASE_A_64(xlT, xrT, elT, attbc, src_pad, dst_pad)
        else:
            cg, es = 32, 1
            sp = _PHASE_A_128(xlT, xrT, elT, attbc, src_pad, dst_pad)
        s2d, gmax = _tc_reduce_s(sp.reshape(cg, E_PAD // 128, 128))
        s1 = s2d.reshape(E_PAD)
        ex, dparts = _sc_phase_b2(s1, gmax, dst_pad)
        rden2 = _tc_rden(dparts.reshape(32, N_PAD // 128, 128))
        rden = rden2.reshape(N_PAD)
        if out_dim == HID:
            h_parts = _PHASE_C_64(xlT, rden, ex, src_pad, dst_pad)
        else:
            h_parts = _PHASE_C_128(xlT, rden, ex, src_pad, dst_pad)
        if l < NUM_LAYERS - 1:
            scale = p['bn_gamma'] / jnp.sqrt(p['bn_var'] + 1e-5)
            shift = p['bn_beta'] - p['bn_mean'] * scale + p['b'] * scale
            hT = _tc_act(h_parts, scale, shift, a2d, es, True)
        else:
            hT128 = h_parts[0]
            bbc = jnp.broadcast_to(p['b'][:, None], (OUT_GAT, 16))
            pooledT = _pool_sc(hT128, batch_pad, bbc)

    out = _tc_mlp(pooledT, params['fc_W'].T, params['fc_b'],
                  params['fc2_W'].T, params['fc2_b'])
    return out.T


# speedup vs baseline: 8.4911x; 1.0028x over previous
import functools

import jax
import jax.numpy as jnp
from jax import lax
from jax.experimental import pallas as pl
from jax.experimental.pallas import tpu as pltpu
from jax.experimental.pallas import tpu_sc as plsc

N = 10000
E = 640000
D_IN = 48
D_EDGE = 16
HID = 64
OUT_GAT = 128
NUM_LAYERS = 6
NUM_GRAPHS = 512

N_PAD = 10240
E_PAD = E + N_PAD
CH = 4064
PAD_NODE = N

_MESH = plsc.VectorSubcoreMesh(core_axis_name="c", subcore_axis_name="s")
_SC_PARAMS = pltpu.CompilerParams(
    needs_layout_passes=False, use_tc_tiling_on_sc=False)

F32 = jnp.float32
I32 = jnp.int32


def _wid():
    return lax.axis_index("s") * 2 + lax.axis_index("c")


def _zero_1d(ref, n):
    def zb(g, c):
        ref[pl.ds(g * 16, 16)] = jnp.zeros((16,), F32)
        return c
    lax.fori_loop(0, n // 16, zb, 0)


@functools.partial(
    pl.kernel,
    out_type=jax.ShapeDtypeStruct((D_IN, N_PAD), F32),
    mesh=_MESH,
    compiler_params=_SC_PARAMS,
    scratch_types=[
        pltpu.VMEM((320, D_IN), F32),
        pltpu.VMEM((D_IN, 320), F32),
    ],
)
def _sc_transpose_x(x_hbm, xt_hbm, row_v, col_v):
    wid = _wid()
    r0 = wid * 320
    pltpu.sync_copy(x_hbm.at[pl.ds(r0, 320)], row_v)
    lanes = jax.lax.iota(I32, 16)

    def body(g, c):
        ridx = g * 16 + lanes
        for cc in range(D_IN):
            v = plsc.load_gather(row_v, [ridx, jnp.full((16,), cc, I32)])
            col_v[cc, pl.ds(g * 16, 16)] = v
        return c

    lax.fori_loop(0, 20, body, 0)
    pltpu.sync_copy(col_v, xt_hbm.at[:, pl.ds(r0, 320)])


@functools.partial(
    pl.kernel,
    out_type=jax.ShapeDtypeStruct((D_EDGE, E), F32),
    mesh=_MESH,
    compiler_params=_SC_PARAMS,
    scratch_types=[
        pltpu.VMEM((2000, D_EDGE), F32),
        pltpu.VMEM((D_EDGE, 2000), F32),
    ],
)
def _sc_transpose_ea(ea_hbm, eat_hbm, row_v, col_v):
    wid = _wid()
    e0 = wid * 20000
    lanes = jax.lax.iota(I32, 16)

    def chunk(k, c0):
        r0 = e0 + k * 2000
        pltpu.sync_copy(ea_hbm.at[pl.ds(r0, 2000)], row_v)

        def body(g, c):
            ridx = g * 16 + lanes
            for cc in range(D_EDGE):
                v = plsc.load_gather(row_v, [ridx, jnp.full((16,), cc, I32)])
                col_v[cc, pl.ds(g * 16, 16)] = v
            return c

        lax.fori_loop(0, 125, body, 0)
        pltpu.sync_copy(col_v, eat_hbm.at[:, pl.ds(r0, 2000)])
        return c0

    lax.fori_loop(0, 10, chunk, 0)


@functools.partial(
    pl.kernel,
    out_type=(
        jax.ShapeDtypeStruct((2, D_EDGE, N_PAD), F32),
        jax.ShapeDtypeStruct((2, N_PAD), F32),
    ),
    mesh=_MESH,
    compiler_params=_SC_PARAMS,
    scratch_types=[
        pltpu.VMEM((N_PAD,), F32),
        pltpu.VMEM((N_PAD,), F32),
        pltpu.VMEM((2000,), F32),
        pltpu.VMEM((2000,), I32),
    ],
)
def _sc_ea_segsum(eat_hbm, dst_hbm, sum_hbm, cnt_hbm, acc_v, cntacc_v, val_v,
                  dst_v):
    wid = _wid()
    col = wid % 16
    half = wid // 16
    e0 = half * (E // 2)
    _zero_1d(acc_v, N_PAD)
    _zero_1d(cntacc_v, N_PAD)
    ones = jnp.ones((16,), F32)

    def chunk(k, c0):
        r0 = e0 + k * 2000
        pltpu.sync_copy(eat_hbm.at[col, pl.ds(r0, 2000)], val_v)
        pltpu.sync_copy(dst_hbm.at[pl.ds(r0, 2000)], dst_v)

        def body(g, c):
            idx = dst_v[pl.ds(g * 16, 16)]
            v = val_v[pl.ds(g * 16, 16)]
            plsc.addupdate_scatter(acc_v, [idx], v)

            @pl.when(col == 0)
            def _():
                plsc.addupdate_scatter(cntacc_v, [idx], ones)

            return c

        lax.fori_loop(0, 125, body, 0)
        return c0

    lax.fori_loop(0, (E // 2) // 2000, chunk, 0)
    pltpu.sync_copy(acc_v, sum_hbm.at[half, col])

    @pl.when(col == 0)
    def _():
        pltpu.sync_copy(cntacc_v, cnt_hbm.at[half])


def _tc_ea_loop_body(sum_ref, cnt_ref, out_ref):
    s = sum_ref[0] + sum_ref[1]
    c = cnt_ref[0] + cnt_ref[1]
    out_ref[...] = s / jnp.maximum(c, 1.0)[None, :]


def _tc_ea_loop(sums, cnts):
    return pl.pallas_call(
        _tc_ea_loop_body,
        out_shape=jax.ShapeDtypeStruct((D_EDGE, N_PAD), F32),
    )(sums, cnts)


def _tc_proj_body(h_ref, wl_ref, wr_ref, xl_ref, xr_ref):
    h = h_ref[...]
    xl_ref[...] = jnp.dot(wl_ref[...], h, preferred_element_type=F32)
    xr_ref[...] = jnp.dot(wr_ref[...], h, preferred_element_type=F32)


def _tc_proj(hT, WlT, WrT):
    out = WlT.shape[0]
    return pl.pallas_call(
        _tc_proj_body,
        out_shape=(
            jax.ShapeDtypeStruct((out, N_PAD), F32),
            jax.ShapeDtypeStruct((out, N_PAD), F32),
        ),
    )(hT, WlT, WrT)


def _tc_act_body(parts_ref, sc_ref, sh_ref, a_ref, out_ref, *, es, bn):
    h = parts_ref[0]
    for i in range(1, es):
        h = h + parts_ref[i]
    if bn:
        h = h * sc_ref[...][:, None] + sh_ref[...][:, None]
        a = a_ref[0, 0]
        h = jnp.where(h >= 0, h, a * h)
    else:
        h = h + sc_ref[...][:, None]
    mask = lax.broadcasted_iota(I32, h.shape, 1) < N
    out_ref[...] = jnp.where(mask, h, 0.0)


def _tc_act(h_parts, scale, shift, a2d, es, bn):
    out = h_parts.shape[1]
    body = functools.partial(_tc_act_body, es=es, bn=bn)
    return pl.pallas_call(
        body,
        out_shape=jax.ShapeDtypeStruct((out, N_PAD), F32),
    )(h_parts, scale, shift, a2d)


def _tc_el_body(we_ref, ea_ref, out_ref):
    out_ref[...] = jnp.dot(we_ref[...], ea_ref[...], preferred_element_type=F32)


def _tc_el(WeT, ea_aT):
    out = WeT.shape[0]
    blk = 5120
    grid = E_PAD // blk
    return pl.pallas_call(
        _tc_el_body,
        grid=(grid,),
        in_specs=[
            pl.BlockSpec((out, D_EDGE), lambda i: (0, 0)),
            pl.BlockSpec((D_EDGE, blk), lambda i: (0, i)),
        ],
        out_specs=pl.BlockSpec((out, blk), lambda i: (0, i)),
        out_shape=jax.ShapeDtypeStruct((out, E_PAD), F32),
    )(WeT, ea_aT)


def _make_sc_phase_a(out_dim, cg, es):
    el_per = E_PAD // es
    CHA = 2032
    nchunk = el_per // CHA

    @functools.partial(
        pl.kernel,
        out_type=jax.ShapeDtypeStruct((cg, E_PAD), F32),
        mesh=_MESH,
        compiler_params=_SC_PARAMS,
        scratch_types=[
            pltpu.VMEM((4, N_PAD), F32),
            pltpu.VMEM((4, N_PAD), F32),
            pltpu.VMEM((4, 16), F32),
            pltpu.VMEM((2, CHA), I32),
            pltpu.VMEM((2, CHA), I32),
            pltpu.VMEM((2, 4, CHA), F32),
            pltpu.VMEM((CHA,), F32),
            pltpu.SemaphoreType.DMA,
            pltpu.SemaphoreType.DMA,
        ],
    )
    def phase_a(xlt_hbm, xrt_hbm, elt_hbm, attbc_hbm, src_hbm, dst_hbm,
                sp_hbm, xl_v, xr_v, att_v, src_v, dst_v, el_v, s_v,
                sem0, sem1):
        wid = _wid()
        cgi = wid % cg
        esi = wid // cg
        e0 = esi * el_per
        c4 = cgi * 4
        pltpu.sync_copy(xlt_hbm.at[pl.ds(c4, 4)], xl_v)
        pltpu.sync_copy(xrt_hbm.at[pl.ds(c4, 4)], xr_v)
        pltpu.sync_copy(attbc_hbm.at[pl.ds(c4, 4)], att_v)
        atts = tuple(att_v[c] for c in range(4))
        sems = (sem0, sem1)

        def copies(k, b):
            r0 = e0 + k * CHA
            return (
                (src_hbm.at[pl.ds(r0, CHA)], src_v.at[b]),
                (dst_hbm.at[pl.ds(r0, CHA)], dst_v.at[b]),
                (elt_hbm.at[pl.ds(c4, 4), pl.ds(r0, CHA)], el_v.at[b]),
            )

        def start_in(k, b):
            for s, d in copies(k, b):
                pltpu.async_copy(s, d, sems[b])

        def wait_in(k, b):
            for s, d in copies(k, b):
                pltpu.make_async_copy(s, d, sems[b]).wait()

        def compute(k, b):
            r0 = e0 + k * CHA

            @plsc.parallel_loop(0, CHA // 16, unroll=8)
            def body(g):
                idxs = src_v[b, pl.ds(g * 16, 16)]
                idxd = dst_v[b, pl.ds(g * 16, 16)]
                acc = jnp.zeros((16,), F32)
                for c in range(4):
                    xa = plsc.load_gather(xl_v.at[c], [idxs])
                    xb = plsc.load_gather(xr_v.at[c], [idxd])
                    m = xa + xb + el_v[b, c, pl.ds(g * 16, 16)]
                    m = jnp.maximum(m, 0.2 * m)
                    acc = acc + atts[c] * m
                s_v[pl.ds(g * 16, 16)] = acc
            pltpu.sync_copy(s_v, sp_hbm.at[cgi, pl.ds(r0, CHA)])

        last = nchunk - 1
        start_in(0, 0)

        def pair(j, c0):
            k0 = 2 * j
            start_in(jnp.minimum(k0 + 1, last), 1)
            wait_in(k0, 0)
            compute(k0, 0)
            start_in(jnp.minimum(k0 + 2, last), 0)
            wait_in(jnp.minimum(k0 + 1, last), 1)
            compute(k0 + 1, 1)
            return c0

        lax.fori_loop(0, nchunk // 2, pair, 0)
        wait_in(last, 0)

    return phase_a


def _tc_reduce_body(sp_ref, s_ref, g_ref):
    blk = sp_ref[...]
    ssum = blk.sum(axis=0)
    s_ref[...] = ssum
    bm = jnp.max(ssum)

    @pl.when(pl.program_id(0) == 0)
    def _():
        g_ref[...] = jnp.full((1, 128), -1e30, F32)

    g_ref[...] = jnp.maximum(g_ref[...], bm)


def _tc_reduce_s(s_parts3):
    cg = s_parts3.shape[0]
    rows = E_PAD // 128
    rblk = 8
    grid = rows // rblk
    return pl.pallas_call(
        _tc_reduce_body,
        grid=(grid,),
        in_specs=[pl.BlockSpec((cg, rblk, 128), lambda i: (0, i, 0))],
        out_specs=(
            pl.BlockSpec((rblk, 128), lambda i: (i, 0)),
            pl.BlockSpec((1, 128), lambda i: (0, 0)),
        ),
        out_shape=(
            jax.ShapeDtypeStruct((rows, 128), F32),
            jax.ShapeDtypeStruct((1, 128), F32),
        ),
    )(s_parts3)


_B2_PER = E_PAD // 32
_B2_CH = 4064


@functools.partial(
    pl.kernel,
    out_type=(
        jax.ShapeDtypeStruct((E_PAD,), F32),
        jax.ShapeDtypeStruct((32, N_PAD), F32),
    ),
    mesh=_MESH,
    compiler_params=_SC_PARAMS,
    scratch_types=[
        pltpu.VMEM((N_PAD,), F32),
        pltpu.VMEM((_B2_CH,), F32),
        pltpu.VMEM((_B2_CH,), I32),
        pltpu.VMEM((16,), F32),
    ],
)
def _sc_phase_b2(s_hbm, g_hbm, dst_hbm, ex_hbm, dp_hbm, den_v, s_v, dst_v,
                 g_v):
    wid = _wid()
    e0 = wid * _B2_PER
    pltpu.sync_copy(g_hbm.at[0, pl.ds(0, 16)], g_v)
    gv = g_v[...]
    _zero_1d(den_v, N_PAD)

    def chunk(k, c0):
        r0 = e0 + k * _B2_CH
        pltpu.sync_copy(s_hbm.at[pl.ds(r0, _B2_CH)], s_v)
        pltpu.sync_copy(dst_hbm.at[pl.ds(r0, _B2_CH)], dst_v)

        def body(g4, c1):
            for u in range(8):
                g = g4 * 8 + u
                sv = s_v[pl.ds(g * 16, 16)]
                ex = jnp.exp(sv - gv)
                s_v[pl.ds(g * 16, 16)] = ex
                idx = dst_v[pl.ds(g * 16, 16)]
                plsc.addupdate_scatter(den_v, [idx], ex)
            return c1

        lax.fori_loop(0, _B2_CH // 128, body, 0)
        pltpu.sync_copy(s_v, ex_hbm.at[pl.ds(r0, _B2_CH)])
        return c0

    lax.fori_loop(0, _B2_PER // _B2_CH, chunk, 0)
    pltpu.sync_copy(den_v, dp_hbm.at[wid])


def _tc_rden_body(dp_ref, out_ref):
    out_ref[...] = 1.0 / (dp_ref[...].sum(axis=0) + 1e-16)


def _tc_rden(dp3):
    return pl.pallas_call(
        _tc_rden_body,
        out_shape=jax.ShapeDtypeStruct((N_PAD // 128, 128), F32),
    )(dp3)


def _make_sc_phase_c(out_dim, cg, es):
    el_per = E_PAD // es
    nchunk = el_per // CH

    @functools.partial(
        pl.kernel,
        out_type=jax.ShapeDtypeStruct((es, out_dim, N_PAD), F32),
        mesh=_MESH,
        compiler_params=_SC_PARAMS,
        scratch_types=[
            pltpu.VMEM((4, N_PAD), F32),
            pltpu.VMEM((4, N_PAD), F32),
            pltpu.VMEM((N_PAD,), F32),
            pltpu.VMEM((2, CH), I32),
            pltpu.VMEM((2, CH), I32),
            pltpu.VMEM((2, CH), F32),
            pltpu.SemaphoreType.DMA,
            pltpu.SemaphoreType.DMA,
        ],
    )
    def phase_c(xlt_hbm, rden_hbm, ex_hbm, src_hbm, dst_hbm, hp_hbm,
                xl_v, h_v, rden_v, src_v, dst_v, ex_v, sem0, sem1):
        wid = _wid()
        cgi = wid % cg
        esi = wid // cg
        e0 = esi * el_per
        c4 = cgi * 4
        sems = (sem0, sem1)
        pltpu.sync_copy(rden_hbm, rden_v)
        pltpu.sync_copy(xlt_hbm.at[pl.ds(c4, 4)], xl_v)

        def zb(g, c):
            z = jnp.zeros((16,), F32)
            for cc in range(4):
                h_v[cc, pl.ds(g * 16, 16)] = z
            return c

        lax.fori_loop(0, N_PAD // 16, zb, 0)

        def copies(k, b):
            r0 = e0 + k * CH
            return (
                (src_hbm.at[pl.ds(r0, CH)], src_v.at[b]),
                (dst_hbm.at[pl.ds(r0, CH)], dst_v.at[b]),
                (ex_hbm.at[pl.ds(r0, CH)], ex_v.at[b]),
            )

        def start_in(k, b):
            for s, d in copies(k, b):
                pltpu.async_copy(s, d, sems[b])

        def wait_in(k, b):
            for s, d in copies(k, b):
                pltpu.make_async_copy(s, d, sems[b]).wait()

        def compute(k, b):
            def body(g4, c1):
                for u in range(8):
                    g = g4 * 8 + u
                    idxs = src_v[b, pl.ds(g * 16, 16)]
                    idxd = dst_v[b, pl.ds(g * 16, 16)]
                    alpha = ex_v[b, pl.ds(g * 16, 16)] * plsc.load_gather(
                        rden_v, [idxd])
                    for c in range(4):
                        xv = plsc.load_gather(xl_v.at[c], [idxs])
                        plsc.addupdate_scatter(h_v.at[c], [idxd], alpha * xv)
                return c1

            lax.fori_loop(0, CH // 128, body, 0)

        last = nchunk - 1
        start_in(0, 0)

        def pair(j, c0):
            k0 = 2 * j
            start_in(jnp.minimum(k0 + 1, last), 1)
            wait_in(k0, 0)
            compute(k0, 0)
            start_in(jnp.minimum(k0 + 2, last), 0)
            wait_in(jnp.minimum(k0 + 1, last), 1)
            compute(k0 + 1, 1)
            return c0

        lax.fori_loop(0, nchunk // 2, pair, 0)
        wait_in(last, 0)
        pltpu.sync_copy(h_v, hp_hbm.at[esi, pl.ds(c4, 4)])

    return phase_c


@functools.partial(
    pl.kernel,
    out_type=jax.ShapeDtypeStruct((OUT_GAT, NUM_GRAPHS), F32),
    mesh=_MESH,
    compiler_params=_SC_PARAMS,
    scratch_types=[
        pltpu.VMEM((N_PAD,), I32),
        pltpu.VMEM((N_PAD,), F32),
        pltpu.VMEM((1024,), F32),
        pltpu.VMEM((16,), F32),
    ],
)
def _pool_sc(hT_hbm, batch_hbm, bbc_hbm, out_hbm, batch_v, col_v, pool_v, b_v):
    wid = _wid()
    pltpu.sync_copy(batch_hbm, batch_v)
    for c in range(4):
        col = wid * 4 + c
        pltpu.sync_copy(hT_hbm.at[col], col_v)
        pltpu.sync_copy(bbc_hbm.at[col], b_v)
        bv = b_v[...]
        _zero_1d(pool_v, 1024)

        def body(i, carry):
            v = col_v[pl.ds(i * 16, 16)] + bv
            idx = batch_v[pl.ds(i * 16, 16)]
            plsc.addupdate_scatter(pool_v, [idx], v)
            return carry

        lax.fori_loop(0, N_PAD // 16, body, 0)
        pltpu.sync_copy(pool_v.at[pl.ds(0, NUM_GRAPHS)], out_hbm.at[col])


def _tc_mlp_body(p_ref, w1_ref, b1_ref, w2_ref, b2_ref, out_ref):
    z = jnp.dot(w1_ref[...], p_ref[...], preferred_element_type=F32)
    z = z + b1_ref[...][:, None]
    z = z * jnp.tanh(jax.nn.softplus(z))
    o = jnp.dot(w2_ref[...], z, preferred_element_type=F32)
    out_ref[...] = o + b2_ref[...][:, None]


def _tc_mlp(pooledT, fcWT, fcb, fc2WT, fc2b):
    return pl.pallas_call(
        _tc_mlp_body,
        out_shape=jax.ShapeDtypeStruct((3, NUM_GRAPHS), F32),
    )(pooledT, fcWT, fcb, fc2WT, fc2b)


_PHASE_A_64 = _make_sc_phase_a(HID, 16, 2)
_PHASE_A_128 = _make_sc_phase_a(OUT_GAT, 32, 1)
_PHASE_C_64 = _make_sc_phase_c(HID, 16, 2)
_PHASE_C_128 = _make_sc_phase_c(OUT_GAT, 32, 1)


def kernel(x, edge_index, edge_attr, batch, params):
    src = edge_index[0]
    dst = edge_index[1]
    loop = jnp.arange(N, dtype=I32)
    padi = jnp.full((N_PAD - N,), PAD_NODE, I32)
    src_pad = jnp.concatenate([src, loop, padi])
    dst_pad = jnp.concatenate([dst, loop, padi])
    batch_pad = jnp.full((N_PAD,), NUM_GRAPHS, I32).at[:N].set(batch)
    x_pad = jnp.zeros((N_PAD, D_IN), F32).at[:N].set(x)

    xT = _sc_transpose_x(x_pad)
    eaT = _sc_transpose_ea(edge_attr)
    sums, cnts = _sc_ea_segsum(eaT, dst)
    ea_loopT = _tc_ea_loop(sums, cnts)
    ea_aT = jnp.concatenate([eaT, ea_loopT], axis=1)

    a2d = jnp.reshape(params['prelu_a'], (1, 1)).astype(F32)
    hT = xT
    for l, p in enumerate(params['layers']):
        out_dim = p['Wl'].shape[1]
        WlT = p['Wl'].T
        WrT = p['Wr'].T
        WeT = p['We'].T
        attbc = jnp.broadcast_to(p['att'][:, None], (out_dim, 16))
        xlT, xrT = _tc_proj(hT, WlT, WrT)
        elT = _tc_el(WeT, ea_aT)
        if out_dim == HID:
            cg, es = 16, 2
            sp = _PHASE_A_64(xlT, xrT, elT, attbc, src_pad, dst_pad)
        else:
            cg, es = 32, 1
            sp = _PHASE_A_128(xlT, xrT, elT, attbc, src_pad, dst_pad)
        s2d, gmax = _tc_reduce_s(sp.reshape(cg, E_PAD // 128, 128))
        s1 = s2d.reshape(E_PAD)
        ex, dparts = _sc_phase_b2(s1, gmax, dst_pad)
        rden2 = _tc_rden(dparts.reshape(32, N_PAD // 128, 128))
        rden = rden2.reshape(N_PAD)
        if out_dim == HID:
            h_parts = _PHASE_C_64(xlT, rden, ex, src_pad, dst_pad)
        else:
            h_parts = _PHASE_C_128(xlT, rden, ex, src_pad, dst_pad)
        if l < NUM_LAYERS - 1:
            scale = p['bn_gamma'] / jnp.sqrt(p['bn_var'] + 1e-5)
            shift = p['bn_beta'] - p['bn_mean'] * scale + p['b'] * scale
            hT = _tc_act(h_parts, scale, shift, a2d, es, True)
        else:
            hT128 = h_parts[0]
            bbc = jnp.broadcast_to(p['b'][:, None], (OUT_GAT, 16))
            pooledT = _pool_sc(hT128, batch_pad, bbc)

    out = _tc_mlp(pooledT, params['fc_W'].T, params['fc_b'],
                  params['fc2_W'].T, params['fc2_b'])
    return out.T
